# Initial kernel scaffold; baseline (speedup 1.0000x reference)
import functools

import jax
import jax.numpy as jnp
from jax import lax
from jax.experimental import pallas as pl
from jax.experimental.pallas import tpu as pltpu
from jax.experimental.pallas import tpu_sc as plsc

_N = 10000
_E = 320000
_D = 128
_G = 64

_NB = 20
_BR = _N // _NB
_NC = 2
_NS = 16
_NW = _NC * _NS
_EPW = _E // _NW
_K = 80
_NCHUNK = _EPW // _K
_RPT = _N // _NS
_ZR = 25
_CW = 16

_PREC = jax.lax.Precision.HIGHEST


def _tc_pre_body(x_ref, wl_ref, wr_ref, b_ref, a_ref, xr_ref):
    xb = x_ref[...]
    a_ref[...] = jnp.dot(xb, wl_ref[...], precision=_PREC,
                         preferred_element_type=jnp.float32)
    xr_ref[...] = jnp.dot(xb, wr_ref[...], precision=_PREC,
                          preferred_element_type=jnp.float32) + b_ref[...]


def _tc_pre(x, wl, wr, b):
    return pl.pallas_call(
        _tc_pre_body,
        grid=(_NB,),
        in_specs=[
            pl.BlockSpec((_BR, _D), lambda i: (i, 0)),
            pl.BlockSpec((_D, _D), lambda i: (0, 0)),
            pl.BlockSpec((_D, _D), lambda i: (0, 0)),
            pl.BlockSpec((1, _D), lambda i: (0, 0)),
        ],
        out_specs=[
            pl.BlockSpec((_BR, _D), lambda i: (i, 0)),
            pl.BlockSpec((_BR, _D), lambda i: (i, 0)),
        ],
        out_shape=[
            jax.ShapeDtypeStruct((_N, _D), jnp.float32),
            jax.ShapeDtypeStruct((_N, _D), jnp.float32),
        ],
    )(x, wl, wr, b)


def _tc_mid_body(p_ref, c_ref, xr_ref, g_ref, bt_ref, wl_ref, wr_ref, b2_ref,
                 a2_ref, hr2_ref):
    pblk = p_ref[0] + p_ref[1]
    cnt = jnp.sum(c_ref[0] + c_ref[1], axis=1, keepdims=True) * (1.0 / _CW)
    h = jnp.maximum(pblk / jnp.maximum(cnt, 1.0) + xr_ref[...], 0.0)
    mu = jnp.mean(h, axis=1, keepdims=True)
    d = h - mu
    var = jnp.mean(d * d, axis=1, keepdims=True)
    h = d * lax.rsqrt(var + 1e-5) * g_ref[...] + bt_ref[...]
    a2_ref[...] = jnp.dot(h, wl_ref[...], precision=_PREC,
                          preferred_element_type=jnp.float32)
    hr2_ref[...] = jnp.dot(h, wr_ref[...], precision=_PREC,
                           preferred_element_type=jnp.float32) + b2_ref[...]


def _tc_mid(p1, c, xr1, gamma, beta, w2l, w2r, b2):
    return pl.pallas_call(
        _tc_mid_body,
        grid=(_NB,),
        in_specs=[
            pl.BlockSpec((_NC, _BR, _D), lambda i: (0, i, 0)),
            pl.BlockSpec((_NC, _BR, _CW), lambda i: (0, i, 0)),
            pl.BlockSpec((_BR, _D), lambda i: (i, 0)),
            pl.BlockSpec((1, _D), lambda i: (0, 0)),
            pl.BlockSpec((1, _D), lambda i: (0, 0)),
            pl.BlockSpec((_D, _D), lambda i: (0, 0)),
            pl.BlockSpec((_D, _D), lambda i: (0, 0)),
            pl.BlockSpec((1, _D), lambda i: (0, 0)),
        ],
        out_specs=[
            pl.BlockSpec((_BR, _D), lambda i: (i, 0)),
            pl.BlockSpec((_BR, _D), lambda i: (i, 0)),
        ],
        out_shape=[
            jax.ShapeDtypeStruct((_N, _D), jnp.float32),
            jax.ShapeDtypeStruct((_N, _D), jnp.float32),
        ],
    )(p1, c, xr1, gamma, beta, w2l, w2r, b2)


def _tc_post_body(p_ref, c_ref, hr_ref, b_ref, wc1_ref, bc1_ref, wc2_ref,
                  bc2_ref, wc3t_ref, bc3_ref, out_ref, acc_ref):
    i = pl.program_id(0)

    @pl.when(i == 0)
    def _():
        acc_ref[...] = jnp.zeros_like(acc_ref)

    pblk = p_ref[0] + p_ref[1]
    cnt = jnp.sum(c_ref[0] + c_ref[1], axis=1, keepdims=True) * (1.0 / _CW)
    h2 = jnp.maximum(pblk / jnp.maximum(cnt, 1.0) + hr_ref[...], 0.0)
    seg = b_ref[0]
    onehot_t = (lax.broadcasted_iota(jnp.int32, (_G, _BR), 0) == seg
                ).astype(jnp.float32)
    acc_ref[...] += jnp.dot(onehot_t, h2, precision=_PREC,
                            preferred_element_type=jnp.float32)

    @pl.when(i == _NB - 1)
    def _():
        pooled = acc_ref[...]
        z = jnp.maximum(jnp.dot(pooled, wc1_ref[...], precision=_PREC,
                                preferred_element_type=jnp.float32)
                        + bc1_ref[...], 0.0)
        z = jnp.maximum(jnp.dot(z, wc2_ref[...], precision=_PREC,
                                preferred_element_type=jnp.float32)
                        + bc2_ref[...], 0.0)
        out_ref[...] = (jnp.sum(z * wc3t_ref[...], axis=1, keepdims=True)
                        + bc3_ref[...])


def _tc_post(p2, c, hr2, batch3, wc1, bc1, wc2, bc2, wc3t, bc3):
    return pl.pallas_call(
        _tc_post_body,
        grid=(_NB,),
        in_specs=[
            pl.BlockSpec((_NC, _BR, _D), lambda i: (0, i, 0)),
            pl.BlockSpec((_NC, _BR, _CW), lambda i: (0, i, 0)),
            pl.BlockSpec((_BR, _D), lambda i: (i, 0)),
            pl.BlockSpec((1, 1, _BR), lambda i: (i, 0, 0)),
            pl.BlockSpec((_D, _G), lambda i: (0, 0)),
            pl.BlockSpec((1, _G), lambda i: (0, 0)),
            pl.BlockSpec((_G, _G), lambda i: (0, 0)),
            pl.BlockSpec((1, _G), lambda i: (0, 0)),
            pl.BlockSpec((1, _G), lambda i: (0, 0)),
            pl.BlockSpec((1, 1), lambda i: (0, 0)),
        ],
        out_specs=pl.BlockSpec((_G, 1), lambda i: (0, 0)),
        out_shape=jax.ShapeDtypeStruct((_G, 1), jnp.float32),
        scratch_shapes=[pltpu.VMEM((_G, _D), jnp.float32)],
    )(p2, c, hr2, batch3, wc1, bc1, wc2, bc2, wc3t, bc3)


def _sc_agg_with_cnt(a, src, dst):
    mesh = plsc.VectorSubcoreMesh(core_axis_name="c", subcore_axis_name="s")

    @functools.partial(
        pl.kernel,
        out_type=[
            jax.ShapeDtypeStruct((_NC, _N, _D), jnp.float32),
            jax.ShapeDtypeStruct((_NC, _N, _CW), jnp.float32),
        ],
        mesh=mesh,
        scratch_types=[
            pltpu.VMEM_SHARED((_N, _D), jnp.float32),
            pltpu.VMEM_SHARED((_N, _CW), jnp.float32),
            pltpu.VMEM((_K,), jnp.int32),
            pltpu.VMEM((_K,), jnp.int32),
            pltpu.VMEM((_K, _D), jnp.float32),
            pltpu.VMEM((_ZR, _D), jnp.float32),
            pltpu.VMEM((_K, _CW), jnp.float32),
            pltpu.VMEM((_ZR, _CW), jnp.float32),
            pltpu.SemaphoreType.DMA,
        ],
    )
    def k(a_hbm, src_hbm, dst_hbm, p_hbm, c_hbm, acc_sh, cnt_sh, src_v, dst_v,
          rows_v, zrows_v, ones_v, zcnt_v, sem):
        c = lax.axis_index("c")
        s = lax.axis_index("s")
        wid = c * _NS + s
        zero16 = jnp.zeros((16,), jnp.float32)
        one16 = jnp.ones((16,), jnp.float32)

        def fill(i, carry):
            for j in range(_D // 16):
                zrows_v[i, pl.ds(j * 16, 16)] = zero16
            zcnt_v[i, pl.ds(0, 16)] = zero16
            return carry

        lax.fori_loop(0, _ZR, fill, 0)

        def fill_ones(i, carry):
            ones_v[i, pl.ds(0, 16)] = one16
            return carry

        lax.fori_loop(0, _K, fill_ones, 0)

        base_r = s * _RPT

        def zero_spmem(r, carry):
            pltpu.sync_copy(zrows_v, acc_sh.at[pl.ds(base_r + r * _ZR, _ZR)])
            pltpu.sync_copy(zcnt_v, cnt_sh.at[pl.ds(base_r + r * _ZR, _ZR)])
            return carry

        lax.fori_loop(0, _RPT // _ZR, zero_spmem, 0)
        plsc.subcore_barrier()

        ebase = wid * _EPW

        def chunk(i, carry):
            off = ebase + i * _K
            pltpu.sync_copy(src_hbm.at[pl.ds(off, _K)], src_v)
            pltpu.sync_copy(dst_hbm.at[pl.ds(off, _K)], dst_v)
            pltpu.async_copy(a_hbm.at[src_v], rows_v, sem).wait()
            pltpu.sync_copy(rows_v, acc_sh.at[dst_v], add=True)
            pltpu.sync_copy(ones_v, cnt_sh.at[dst_v], add=True)
            return carry

        lax.fori_loop(0, _NCHUNK, chunk, 0)
        plsc.subcore_barrier()

        pltpu.sync_copy(acc_sh.at[pl.ds(base_r, _RPT)],
                        p_hbm.at[c, pl.ds(base_r, _RPT)])
        pltpu.sync_copy(cnt_sh.at[pl.ds(base_r, _RPT)],
                        c_hbm.at[c, pl.ds(base_r, _RPT)])

    return k(a, src, dst)


def _sc_agg(a, src, dst):
    mesh = plsc.VectorSubcoreMesh(core_axis_name="c", subcore_axis_name="s")

    @functools.partial(
        pl.kernel,
        out_type=jax.ShapeDtypeStruct((_NC, _N, _D), jnp.float32),
        mesh=mesh,
        scratch_types=[
            pltpu.VMEM_SHARED((_N, _D), jnp.float32),
            pltpu.VMEM((_K,), jnp.int32),
            pltpu.VMEM((_K,), jnp.int32),
            pltpu.VMEM((_K, _D), jnp.float32),
            pltpu.VMEM((_ZR, _D), jnp.float32),
            pltpu.SemaphoreType.DMA,
        ],
    )
    def k(a_hbm, src_hbm, dst_hbm, p_hbm, acc_sh, src_v, dst_v, rows_v,
          zrows_v, sem):
        c = lax.axis_index("c")
        s = lax.axis_index("s")
        wid = c * _NS + s
        zero16 = jnp.zeros((16,), jnp.float32)

        def fill(i, carry):
            for j in range(_D // 16):
                zrows_v[i, pl.ds(j * 16, 16)] = zero16
            return carry

        lax.fori_loop(0, _ZR, fill, 0)

        base_r = s * _RPT

        def zero_spmem(r, carry):
            pltpu.sync_copy(zrows_v, acc_sh.at[pl.ds(base_r + r * _ZR, _ZR)])
            return carry

        lax.fori_loop(0, _RPT // _ZR, zero_spmem, 0)
        plsc.subcore_barrier()

        ebase = wid * _EPW

        def chunk(i, carry):
            off = ebase + i * _K
            pltpu.sync_copy(src_hbm.at[pl.ds(off, _K)], src_v)
            pltpu.sync_copy(dst_hbm.at[pl.ds(off, _K)], dst_v)
            pltpu.async_copy(a_hbm.at[src_v], rows_v, sem).wait()
            pltpu.sync_copy(rows_v, acc_sh.at[dst_v], add=True)
            return carry

        lax.fori_loop(0, _NCHUNK, chunk, 0)
        plsc.subcore_barrier()

        pltpu.sync_copy(acc_sh.at[pl.ds(base_r, _RPT)],
                        p_hbm.at[c, pl.ds(base_r, _RPT)])

    return k(a, src, dst)


def kernel(x, edge_index, batch, W1l, W1r, b1, gamma, beta, W2l, W2r, b2,
           Wc1, bc1, Wc2, bc2, Wc3, bc3):
    src = edge_index[0]
    dst = edge_index[1]

    a1, xr1 = _tc_pre(x, W1l, W1r, b1.reshape(1, _D))
    p1, cpart = _sc_agg_with_cnt(a1, src, dst)
    a2, hr2 = _tc_mid(p1, cpart, xr1, gamma.reshape(1, _D),
                      beta.reshape(1, _D), W2l, W2r, b2.reshape(1, _D))
    p2 = _sc_agg(a2, src, dst)
    pred = _tc_post(p2, cpart, hr2, batch.reshape(_NB, 1, _BR),
                    Wc1, bc1.reshape(1, _G), Wc2, bc2.reshape(1, _G),
                    Wc3.reshape(1, _G), bc3.reshape(1, 1))
    return pred


# speedup vs baseline: 4.3975x; 4.3975x over previous
import functools

import jax
import jax.numpy as jnp
from jax import lax
from jax.experimental import pallas as pl
from jax.experimental.pallas import tpu as pltpu
from jax.experimental.pallas import tpu_sc as plsc

_N = 10000
_E = 320000
_D = 128
_G = 64

_NP = 10240
_NB = 10
_BR = _NP // _NB
_NC = 2
_NS = 16
_NW = _NC * _NS
_EPW = _E // _NW
_K = 80
_NCHUNK = _EPW // _K
_RB = _NP // _NS
_ZR = 64
_NZ = _RB // _ZR
_CW = 16

_PREC = jax.lax.Precision.HIGHEST


def _tc_pre_body(x_ref, wl_ref, wr_ref, b_ref, a_ref, xr_ref):
    xb = x_ref[...]
    a_ref[...] = jnp.dot(xb, wl_ref[...], precision=_PREC,
                         preferred_element_type=jnp.float32)
    xr_ref[...] = jnp.dot(xb, wr_ref[...], precision=_PREC,
                          preferred_element_type=jnp.float32) + b_ref[...]


def _tc_pre(x, wl, wr, b):
    return pl.pallas_call(
        _tc_pre_body,
        grid=(_NB,),
        in_specs=[
            pl.BlockSpec((_BR, _D), lambda i: (i, 0)),
            pl.BlockSpec((_D, _D), lambda i: (0, 0)),
            pl.BlockSpec((_D, _D), lambda i: (0, 0)),
            pl.BlockSpec((1, _D), lambda i: (0, 0)),
        ],
        out_specs=[
            pl.BlockSpec((_BR, _D), lambda i: (i, 0)),
            pl.BlockSpec((_BR, _D), lambda i: (i, 0)),
        ],
        out_shape=[
            jax.ShapeDtypeStruct((_NP, _D), jnp.float32),
            jax.ShapeDtypeStruct((_NP, _D), jnp.float32),
        ],
    )(x, wl, wr, b)


def _tc_mid_body(p_ref, c_ref, xr_ref, g_ref, bt_ref, wl_ref, wr_ref, b2_ref,
                 a2_ref, hr2_ref):
    pblk = p_ref[0] + p_ref[1]
    cnt = jnp.sum(c_ref[0] + c_ref[1], axis=1, keepdims=True) * (1.0 / _D)
    h = jnp.maximum(pblk / jnp.maximum(cnt, 1.0) + xr_ref[...], 0.0)
    mu = jnp.mean(h, axis=1, keepdims=True)
    d = h - mu
    var = jnp.mean(d * d, axis=1, keepdims=True)
    h = d * lax.rsqrt(var + 1e-5) * g_ref[...] + bt_ref[...]
    a2_ref[...] = jnp.dot(h, wl_ref[...], precision=_PREC,
                          preferred_element_type=jnp.float32)
    hr2_ref[...] = jnp.dot(h, wr_ref[...], precision=_PREC,
                           preferred_element_type=jnp.float32) + b2_ref[...]


def _tc_mid(p1, c, xr1, gamma, beta, w2l, w2r, b2):
    return pl.pallas_call(
        _tc_mid_body,
        grid=(_NB,),
        in_specs=[
            pl.BlockSpec((_NC, _BR, _D), lambda i: (0, i, 0)),
            pl.BlockSpec((_NC, _BR, _D), lambda i: (0, i, 0)),
            pl.BlockSpec((_BR, _D), lambda i: (i, 0)),
            pl.BlockSpec((1, _D), lambda i: (0, 0)),
            pl.BlockSpec((1, _D), lambda i: (0, 0)),
            pl.BlockSpec((_D, _D), lambda i: (0, 0)),
            pl.BlockSpec((_D, _D), lambda i: (0, 0)),
            pl.BlockSpec((1, _D), lambda i: (0, 0)),
        ],
        out_specs=[
            pl.BlockSpec((_BR, _D), lambda i: (i, 0)),
            pl.BlockSpec((_BR, _D), lambda i: (i, 0)),
        ],
        out_shape=[
            jax.ShapeDtypeStruct((_NP, _D), jnp.float32),
            jax.ShapeDtypeStruct((_NP, _D), jnp.float32),
        ],
    )(p1, c, xr1, gamma, beta, w2l, w2r, b2)


def _tc_post_body(p_ref, c_ref, hr_ref, b_ref, wc1_ref, bc1_ref, wc2_ref,
                  bc2_ref, wc3t_ref, bc3_ref, out_ref, acc_ref):
    i = pl.program_id(0)

    @pl.when(i == 0)
    def _():
        acc_ref[...] = jnp.zeros_like(acc_ref)

    pblk = p_ref[0] + p_ref[1]
    cnt = jnp.sum(c_ref[0] + c_ref[1], axis=1, keepdims=True) * (1.0 / _D)
    h2 = jnp.maximum(pblk / jnp.maximum(cnt, 1.0) + hr_ref[...], 0.0)
    seg = b_ref[0]
    onehot_t = (lax.broadcasted_iota(jnp.int32, (_G, _BR), 0) == seg
                ).astype(jnp.float32)
    acc_ref[...] += jnp.dot(onehot_t, h2, precision=_PREC,
                            preferred_element_type=jnp.float32)

    @pl.when(i == _NB - 1)
    def _():
        pooled = acc_ref[...]
        z = jnp.maximum(jnp.dot(pooled, wc1_ref[...], precision=_PREC,
                                preferred_element_type=jnp.float32)
                        + bc1_ref[...], 0.0)
        z = jnp.maximum(jnp.dot(z, wc2_ref[...], precision=_PREC,
                                preferred_element_type=jnp.float32)
                        + bc2_ref[...], 0.0)
        out_ref[...] = (jnp.sum(z * wc3t_ref[...], axis=1, keepdims=True)
                        + bc3_ref[...])


def _tc_post(p2, c, hr2, batch3, wc1, bc1, wc2, bc2, wc3t, bc3):
    return pl.pallas_call(
        _tc_post_body,
        grid=(_NB,),
        in_specs=[
            pl.BlockSpec((_NC, _BR, _D), lambda i: (0, i, 0)),
            pl.BlockSpec((_NC, _BR, _D), lambda i: (0, i, 0)),
            pl.BlockSpec((_BR, _D), lambda i: (i, 0)),
            pl.BlockSpec((1, 1, _BR), lambda i: (i, 0, 0)),
            pl.BlockSpec((_D, _G), lambda i: (0, 0)),
            pl.BlockSpec((1, _G), lambda i: (0, 0)),
            pl.BlockSpec((_G, _G), lambda i: (0, 0)),
            pl.BlockSpec((1, _G), lambda i: (0, 0)),
            pl.BlockSpec((1, _G), lambda i: (0, 0)),
            pl.BlockSpec((1, 1), lambda i: (0, 0)),
        ],
        out_specs=pl.BlockSpec((_G, 1), lambda i: (0, 0)),
        out_shape=jax.ShapeDtypeStruct((_G, 1), jnp.float32),
        scratch_shapes=[pltpu.VMEM((_G, _D), jnp.float32)],
    )(p2, c, hr2, batch3, wc1, bc1, wc2, bc2, wc3t, bc3)


def _sc_cnt(dst):
    mesh = plsc.VectorSubcoreMesh(core_axis_name="c", subcore_axis_name="s")

    @functools.partial(
        pl.kernel,
        out_type=jax.ShapeDtypeStruct((_NC, _NP, _D), jnp.float32),
        mesh=mesh,
        scratch_types=[
            pltpu.VMEM_SHARED((_NP, _D), jnp.float32),
            pltpu.VMEM((_K,), jnp.int32),
            pltpu.VMEM((_K, _D), jnp.float32),
            pltpu.VMEM((_ZR, _D), jnp.float32),
        ],
    )
    def k(dst_hbm, c_hbm, acc_sh, dst_v, ones_v, zrows_v):
        c = lax.axis_index("c")
        s = lax.axis_index("s")
        wid = c * _NS + s
        zero16 = jnp.zeros((16,), jnp.float32)
        one16 = jnp.ones((16,), jnp.float32)

        def fill(i, carry):
            for j in range(_D // 16):
                zrows_v[i, pl.ds(j * 16, 16)] = zero16
            return carry

        lax.fori_loop(0, _ZR, fill, 0)

        def fill_ones(i, carry):
            for j in range(_D // 16):
                ones_v[i, pl.ds(j * 16, 16)] = one16
            return carry

        lax.fori_loop(0, _K, fill_ones, 0)

        base_r = s * _RB

        def zero_spmem(r, carry):
            pltpu.sync_copy(zrows_v, acc_sh.at[pl.ds(base_r + r * _ZR, _ZR)])
            return carry

        lax.fori_loop(0, _NZ, zero_spmem, 0)
        plsc.subcore_barrier()

        ebase = wid * _EPW

        def chunk(i, carry):
            off = ebase + i * _K
            pltpu.sync_copy(dst_hbm.at[pl.ds(off, _K)], dst_v)
            pltpu.sync_copy(ones_v, acc_sh.at[dst_v], add=True)
            return carry

        lax.fori_loop(0, _NCHUNK, chunk, 0)
        plsc.subcore_barrier()

        pltpu.sync_copy(acc_sh.at[pl.ds(base_r, _RB)],
                        c_hbm.at[c, pl.ds(base_r, _RB)])

    return k(dst)


def _sc_agg(a, src, dst):
    mesh = plsc.VectorSubcoreMesh(core_axis_name="c", subcore_axis_name="s")

    @functools.partial(
        pl.kernel,
        out_type=jax.ShapeDtypeStruct((_NC, _NP, _D), jnp.float32),
        mesh=mesh,
        scratch_types=[
            pltpu.VMEM_SHARED((_NP, _D), jnp.float32),
            pltpu.VMEM((_K,), jnp.int32),
            pltpu.VMEM((_K,), jnp.int32),
            pltpu.VMEM((_K, _D), jnp.float32),
            pltpu.VMEM((_ZR, _D), jnp.float32),
            pltpu.SemaphoreType.DMA,
        ],
    )
    def k(a_hbm, src_hbm, dst_hbm, p_hbm, acc_sh, src_v, dst_v, rows_v,
          zrows_v, sem):
        c = lax.axis_index("c")
        s = lax.axis_index("s")
        wid = c * _NS + s
        zero16 = jnp.zeros((16,), jnp.float32)

        def fill(i, carry):
            for j in range(_D // 16):
                zrows_v[i, pl.ds(j * 16, 16)] = zero16
            return carry

        lax.fori_loop(0, _ZR, fill, 0)

        base_r = s * _RB

        def zero_spmem(r, carry):
            pltpu.sync_copy(zrows_v, acc_sh.at[pl.ds(base_r + r * _ZR, _ZR)])
            return carry

        lax.fori_loop(0, _NZ, zero_spmem, 0)
        plsc.subcore_barrier()

        ebase = wid * _EPW

        def chunk(i, carry):
            off = ebase + i * _K
            pltpu.sync_copy(src_hbm.at[pl.ds(off, _K)], src_v)
            pltpu.sync_copy(dst_hbm.at[pl.ds(off, _K)], dst_v)
            pltpu.async_copy(a_hbm.at[src_v], rows_v, sem).wait()
            pltpu.sync_copy(rows_v, acc_sh.at[dst_v], add=True)
            return carry

        lax.fori_loop(0, _NCHUNK, chunk, 0)
        plsc.subcore_barrier()

        pltpu.sync_copy(acc_sh.at[pl.ds(base_r, _RB)],
                        p_hbm.at[c, pl.ds(base_r, _RB)])

    return k(a, src, dst)


def kernel(x, edge_index, batch, W1l, W1r, b1, gamma, beta, W2l, W2r, b2,
           Wc1, bc1, Wc2, bc2, Wc3, bc3):
    src = edge_index[0]
    dst = edge_index[1]
    xp = jnp.pad(x, ((0, _NP - _N), (0, 0)))
    batchp = jnp.pad(batch, (0, _NP - _N), constant_values=_G)

    a1, xr1 = _tc_pre(xp, W1l, W1r, b1.reshape(1, _D))
    cpart = _sc_cnt(dst)
    p1 = _sc_agg(a1, src, dst)
    a2, hr2 = _tc_mid(p1, cpart, xr1, gamma.reshape(1, _D),
                      beta.reshape(1, _D), W2l, W2r, b2.reshape(1, _D))
    p2 = _sc_agg(a2, src, dst)
    pred = _tc_post(p2, cpart, hr2, batchp.reshape(_NB, 1, _BR),
                    Wc1, bc1.reshape(1, _G), Wc2, bc2.reshape(1, _G),
                    Wc3.reshape(1, _G), bc3.reshape(1, 1))
    return pred


# speedup vs baseline: 8.5572x; 1.9460x over previous
import functools

import jax
import jax.numpy as jnp
from jax import lax
from jax.experimental import pallas as pl
from jax.experimental.pallas import tpu as pltpu
from jax.experimental.pallas import tpu_sc as plsc

_N = 10000
_E = 320000
_D = 128
_G = 64

_NP = 10240
_NB = 10
_BR = _NP // _NB
_NC = 2
_NS = 16
_NW = _NC * _NS
_EPW = _E // _NW
_K = 80
_NCHUNK = _EPW // _K
_RB = _NP // _NS
_ZR = 64
_NZ = _RB // _ZR

_HI = jax.lax.Precision.HIGHEST


def _tc_mid_body(p_ref, c_ref, x_ref, g_ref, bt_ref, w1l_ref, w1r_ref, b1_ref,
                 w2r_ref, b2_ref, h_ref, hr2_ref):
    cnt = jnp.sum(c_ref[0] + c_ref[1], axis=1, keepdims=True) * (1.0 / _D)
    mean1 = (p_ref[0] + p_ref[1]) / jnp.maximum(cnt, 1.0)
    h = jnp.maximum(jnp.dot(mean1, w1l_ref[...],
                            preferred_element_type=jnp.float32)
                    + jnp.dot(x_ref[...], w1r_ref[...],
                              preferred_element_type=jnp.float32)
                    + b1_ref[...], 0.0)
    mu = jnp.mean(h, axis=1, keepdims=True)
    d = h - mu
    var = jnp.mean(d * d, axis=1, keepdims=True)
    h = d / jnp.sqrt(var + 1e-5) * g_ref[...] + bt_ref[...]
    h_ref[...] = h
    hr2_ref[...] = jnp.dot(h, w2r_ref[...],
                           preferred_element_type=jnp.float32) + b2_ref[...]


def _tc_mid(p1, c, xp, gamma, beta, w1l, w1r, b1, w2r, b2):
    return pl.pallas_call(
        _tc_mid_body,
        grid=(_NB,),
        in_specs=[
            pl.BlockSpec((_NC, _BR, _D), lambda i: (0, i, 0)),
            pl.BlockSpec((_NC, _BR, _D), lambda i: (0, i, 0)),
            pl.BlockSpec((_BR, _D), lambda i: (i, 0)),
            pl.BlockSpec((1, _D), lambda i: (0, 0)),
            pl.BlockSpec((1, _D), lambda i: (0, 0)),
            pl.BlockSpec((_D, _D), lambda i: (0, 0)),
            pl.BlockSpec((_D, _D), lambda i: (0, 0)),
            pl.BlockSpec((1, _D), lambda i: (0, 0)),
            pl.BlockSpec((_D, _D), lambda i: (0, 0)),
            pl.BlockSpec((1, _D), lambda i: (0, 0)),
        ],
        out_specs=[
            pl.BlockSpec((_BR, _D), lambda i: (i, 0)),
            pl.BlockSpec((_BR, _D), lambda i: (i, 0)),
        ],
        out_shape=[
            jax.ShapeDtypeStruct((_NP, _D), jnp.float32),
            jax.ShapeDtypeStruct((_NP, _D), jnp.float32),
        ],
    )(p1, c, xp, gamma, beta, w1l, w1r, b1, w2r, b2)


def _tc_post_body(p_ref, c_ref, hr_ref, b_ref, w2l_ref, wc1_ref, bc1_ref,
                  wc2_ref, bc2_ref, wc3_ref, bc3_ref, out_ref, acc_ref):
    i = pl.program_id(0)

    @pl.when(i == 0)
    def _():
        acc_ref[...] = jnp.zeros_like(acc_ref)

    cnt = jnp.sum(c_ref[0] + c_ref[1], axis=1, keepdims=True) * (1.0 / _D)
    mean2 = (p_ref[0] + p_ref[1]) / jnp.maximum(cnt, 1.0)
    h2 = jnp.maximum(jnp.dot(mean2, w2l_ref[...],
                             preferred_element_type=jnp.float32)
                     + hr_ref[...], 0.0)
    seg = b_ref[0]
    onehot_t = (lax.broadcasted_iota(jnp.int32, (_G, _BR), 0) == seg
                ).astype(jnp.float32)
    acc_ref[...] += jnp.dot(onehot_t, h2, precision=_HI,
                            preferred_element_type=jnp.float32)

    @pl.when(i == _NB - 1)
    def _():
        pooled = acc_ref[...]
        z = jnp.maximum(jnp.dot(pooled, wc1_ref[...],
                                preferred_element_type=jnp.float32)
                        + bc1_ref[...], 0.0)
        z = jnp.maximum(jnp.dot(z, wc2_ref[...],
                                preferred_element_type=jnp.float32)
                        + bc2_ref[...], 0.0)
        out_ref[...] = jnp.dot(z, wc3_ref[...],
                               preferred_element_type=jnp.float32) + bc3_ref[...]


def _tc_post(p2, c, hr2, batch3, w2l, wc1, bc1, wc2, bc2, wc3, bc3):
    return pl.pallas_call(
        _tc_post_body,
        grid=(_NB,),
        in_specs=[
            pl.BlockSpec((_NC, _BR, _D), lambda i: (0, i, 0)),
            pl.BlockSpec((_NC, _BR, _D), lambda i: (0, i, 0)),
            pl.BlockSpec((_BR, _D), lambda i: (i, 0)),
            pl.BlockSpec((1, 1, _BR), lambda i: (i, 0, 0)),
            pl.BlockSpec((_D, _D), lambda i: (0, 0)),
            pl.BlockSpec((_D, _G), lambda i: (0, 0)),
            pl.BlockSpec((1, _G), lambda i: (0, 0)),
            pl.BlockSpec((_G, _G), lambda i: (0, 0)),
            pl.BlockSpec((1, _G), lambda i: (0, 0)),
            pl.BlockSpec((_G, 1), lambda i: (0, 0)),
            pl.BlockSpec((1, 1), lambda i: (0, 0)),
        ],
        out_specs=pl.BlockSpec((_G, 1), lambda i: (0, 0)),
        out_shape=jax.ShapeDtypeStruct((_G, 1), jnp.float32),
        scratch_shapes=[pltpu.VMEM((_G, _D), jnp.float32)],
    )(p2, c, hr2, batch3, w2l, wc1, bc1, wc2, bc2, wc3, bc3)


def _sc_cnt(dst):
    mesh = plsc.VectorSubcoreMesh(core_axis_name="c", subcore_axis_name="s")

    @functools.partial(
        pl.kernel,
        out_type=jax.ShapeDtypeStruct((_NC, _NP, _D), jnp.float32),
        mesh=mesh,
        scratch_types=[
            pltpu.VMEM_SHARED((_NP, _D), jnp.float32),
            pltpu.VMEM((2, _K), jnp.int32),
            pltpu.VMEM((_K, _D), jnp.float32),
            pltpu.VMEM((_ZR, _D), jnp.float32),
            pltpu.SemaphoreType.DMA,
            pltpu.SemaphoreType.DMA,
        ],
    )
    def k(dst_hbm, c_hbm, acc_sh, dstb, ones_v, zrows_v, isem0, isem1):
        c = lax.axis_index("c")
        s = lax.axis_index("s")
        wid = c * _NS + s
        zero16 = jnp.zeros((16,), jnp.float32)
        one16 = jnp.ones((16,), jnp.float32)
        isems = (isem0, isem1)

        def fill(i, carry):
            for j in range(_D // 16):
                zrows_v[i, pl.ds(j * 16, 16)] = zero16
            return carry

        lax.fori_loop(0, _ZR, fill, 0)

        def fill_ones(i, carry):
            for j in range(_D // 16):
                ones_v[i, pl.ds(j * 16, 16)] = one16
            return carry

        lax.fori_loop(0, _K, fill_ones, 0)

        base_r = s * _RB

        def zero_spmem(r, carry):
            pltpu.sync_copy(zrows_v, acc_sh.at[pl.ds(base_r + r * _ZR, _ZR)])
            return carry

        lax.fori_loop(0, _NZ, zero_spmem, 0)
        plsc.subcore_barrier()

        ebase = wid * _EPW

        def issue_idx(chunk_i, b):
            off = ebase + chunk_i * _K
            pltpu.async_copy(dst_hbm.at[pl.ds(off, _K)], dstb.at[b], isems[b])

        def wait_idx(b):
            pltpu.make_async_copy(dst_hbm.at[pl.ds(0, _K)], dstb.at[b],
                                  isems[b]).wait()

        def step(i, b):
            wait_idx(b)
            pltpu.sync_copy(ones_v, acc_sh.at[dstb.at[b]], add=True)
            issue_idx(jnp.minimum(i + 2, _NCHUNK - 1), b)

        issue_idx(0, 0)
        issue_idx(1, 1)

        def outer(j, carry):
            step(2 * j, 0)
            step(2 * j + 1, 1)
            return carry

        lax.fori_loop(0, (_NCHUNK - 1) // 2, outer, 0)
        wait_idx((_NCHUNK - 1) % 2)
        pltpu.sync_copy(ones_v, acc_sh.at[dstb.at[(_NCHUNK - 1) % 2]], add=True)
        wait_idx(_NCHUNK % 2)
        plsc.subcore_barrier()

        pltpu.sync_copy(acc_sh.at[pl.ds(base_r, _RB)],
                        c_hbm.at[c, pl.ds(base_r, _RB)])

    return k(dst)


def _sc_agg(a, src, dst):
    mesh = plsc.VectorSubcoreMesh(core_axis_name="c", subcore_axis_name="s")

    @functools.partial(
        pl.kernel,
        out_type=jax.ShapeDtypeStruct((_NC, _NP, _D), jnp.float32),
        mesh=mesh,
        scratch_types=[
            pltpu.VMEM_SHARED((_NP, _D), jnp.float32),
            pltpu.VMEM((2, _K), jnp.int32),
            pltpu.VMEM((2, _K), jnp.int32),
            pltpu.VMEM((2, _K, _D), jnp.float32),
            pltpu.VMEM((_ZR, _D), jnp.float32),
            pltpu.SemaphoreType.DMA,
            pltpu.SemaphoreType.DMA,
            pltpu.SemaphoreType.DMA,
            pltpu.SemaphoreType.DMA,
        ],
    )
    def k(a_hbm, src_hbm, dst_hbm, p_hbm, acc_sh, srcb, dstb, rowsb,
          zrows_v, isem0, isem1, gsem0, gsem1):
        c = lax.axis_index("c")
        s = lax.axis_index("s")
        wid = c * _NS + s
        zero16 = jnp.zeros((16,), jnp.float32)
        isems = (isem0, isem1)
        gsems = (gsem0, gsem1)

        def fill(i, carry):
            for j in range(_D // 16):
                zrows_v[i, pl.ds(j * 16, 16)] = zero16
            return carry

        lax.fori_loop(0, _ZR, fill, 0)

        base_r = s * _RB

        def zero_spmem(r, carry):
            pltpu.sync_copy(zrows_v, acc_sh.at[pl.ds(base_r + r * _ZR, _ZR)])
            return carry

        lax.fori_loop(0, _NZ, zero_spmem, 0)
        plsc.subcore_barrier()

        ebase = wid * _EPW

        def issue_idx(chunk_i, b):
            off = ebase + chunk_i * _K
            pltpu.async_copy(src_hbm.at[pl.ds(off, _K)], srcb.at[b], isems[b])
            pltpu.async_copy(dst_hbm.at[pl.ds(off, _K)], dstb.at[b], isems[b])

        def wait_idx(b):
            pltpu.make_async_copy(src_hbm.at[pl.ds(0, _K)], srcb.at[b],
                                  isems[b]).wait()
            pltpu.make_async_copy(dst_hbm.at[pl.ds(0, _K)], dstb.at[b],
                                  isems[b]).wait()

        def issue_gather(b):
            pltpu.async_copy(a_hbm.at[srcb.at[b]], rowsb.at[b], gsems[b])

        def wait_gather(b):
            pltpu.make_async_copy(a_hbm.at[srcb.at[b]], rowsb.at[b],
                                  gsems[b]).wait()

        def consume(b):
            wait_gather(b)
            pltpu.sync_copy(rowsb.at[b], acc_sh.at[dstb.at[b]], add=True)

        issue_idx(0, 0)
        wait_idx(0)
        issue_gather(0)
        issue_idx(1, 1)

        def step(i, b):
            q = 1 - b
            wait_idx(q)
            issue_gather(q)
            consume(b)
            nxt = jnp.minimum(i + 2, _NCHUNK - 1)
            issue_idx(nxt, b)

        def outer(j, carry):
            step(2 * j, 0)
            step(2 * j + 1, 1)
            return carry

        lax.fori_loop(0, (_NCHUNK - 1) // 2, outer, 0)
        consume((_NCHUNK - 1) % 2)
        wait_idx((_NCHUNK - 2) % 2)
        plsc.subcore_barrier()

        pltpu.sync_copy(acc_sh.at[pl.ds(base_r, _RB)],
                        p_hbm.at[c, pl.ds(base_r, _RB)])

    return k(a, src, dst)


def kernel(x, edge_index, batch, W1l, W1r, b1, gamma, beta, W2l, W2r, b2,
           Wc1, bc1, Wc2, bc2, Wc3, bc3):
    src = edge_index[0]
    dst = edge_index[1]
    xp = jnp.pad(x, ((0, _NP - _N), (0, 0)))
    batchp = jnp.pad(batch, (0, _NP - _N), constant_values=_G)

    cpart = _sc_cnt(dst)
    p1 = _sc_agg(xp, src, dst)
    h, hr2 = _tc_mid(p1, cpart, xp, gamma.reshape(1, _D), beta.reshape(1, _D),
                     W1l, W1r, b1.reshape(1, _D), W2r, b2.reshape(1, _D))
    p2 = _sc_agg(h, src, dst)
    pred = _tc_post(p2, cpart, hr2, batchp.reshape(_NB, 1, _BR),
                    W2l, Wc1, bc1.reshape(1, _G), Wc2, bc2.reshape(1, _G),
                    Wc3, bc3.reshape(1, 1))
    return pred


# speedup vs baseline: 9.8225x; 1.1479x over previous
import functools

import jax
import jax.numpy as jnp
from jax import lax
from jax.experimental import pallas as pl
from jax.experimental.pallas import tpu as pltpu
from jax.experimental.pallas import tpu_sc as plsc

_N = 10000
_E = 320000
_D = 128
_G = 64

_NP = 10240
_NB = 10
_BR = _NP // _NB
_NC = 2
_NS = 16
_NW = _NC * _NS
_EPW = _E // _NW
_K = 80
_NCHUNK = _EPW // _K
_RB = _NP // _NS
_ZR = 64
_NZ = _RB // _ZR

_HI = jax.lax.Precision.HIGHEST


def _tc_mid_body(p_ref, c_ref, x_ref, g_ref, bt_ref, w1l_ref, w1r_ref, b1_ref,
                 w2r_ref, b2_ref, h_ref, hr2_ref):
    cnt = jnp.sum(c_ref[0] + c_ref[1], axis=1, keepdims=True) * (1.0 / _D)
    mean1 = (p_ref[0] + p_ref[1]) / jnp.maximum(cnt, 1.0)
    h = jnp.maximum(jnp.dot(mean1, w1l_ref[...],
                            preferred_element_type=jnp.float32)
                    + jnp.dot(x_ref[...], w1r_ref[...],
                              preferred_element_type=jnp.float32)
                    + b1_ref[...], 0.0)
    mu = jnp.mean(h, axis=1, keepdims=True)
    d = h - mu
    var = jnp.mean(d * d, axis=1, keepdims=True)
    h = d / jnp.sqrt(var + 1e-5) * g_ref[...] + bt_ref[...]
    h_ref[...] = h
    hr2_ref[...] = jnp.dot(h, w2r_ref[...],
                           preferred_element_type=jnp.float32) + b2_ref[...]


def _tc_mid(p1, c, xp, gamma, beta, w1l, w1r, b1, w2r, b2):
    return pl.pallas_call(
        _tc_mid_body,
        grid=(_NB,),
        in_specs=[
            pl.BlockSpec((_NC, _BR, _D), lambda i: (0, i, 0)),
            pl.BlockSpec((_NC, _BR, _D), lambda i: (0, i, 0)),
            pl.BlockSpec((_BR, _D), lambda i: (i, 0)),
            pl.BlockSpec((1, _D), lambda i: (0, 0)),
            pl.BlockSpec((1, _D), lambda i: (0, 0)),
            pl.BlockSpec((_D, _D), lambda i: (0, 0)),
            pl.BlockSpec((_D, _D), lambda i: (0, 0)),
            pl.BlockSpec((1, _D), lambda i: (0, 0)),
            pl.BlockSpec((_D, _D), lambda i: (0, 0)),
            pl.BlockSpec((1, _D), lambda i: (0, 0)),
        ],
        out_specs=[
            pl.BlockSpec((_BR, _D), lambda i: (i, 0)),
            pl.BlockSpec((_BR, _D), lambda i: (i, 0)),
        ],
        out_shape=[
            jax.ShapeDtypeStruct((_NP, _D), jnp.float32),
            jax.ShapeDtypeStruct((_NP, _D), jnp.float32),
        ],
    )(p1, c, xp, gamma, beta, w1l, w1r, b1, w2r, b2)


def _tc_post_body(p_ref, c_ref, hr_ref, b_ref, w2l_ref, wc1_ref, bc1_ref,
                  wc2_ref, bc2_ref, wc3_ref, bc3_ref, out_ref, acc_ref):
    i = pl.program_id(0)

    @pl.when(i == 0)
    def _():
        acc_ref[...] = jnp.zeros_like(acc_ref)

    cnt = jnp.sum(c_ref[0] + c_ref[1], axis=1, keepdims=True) * (1.0 / _D)
    mean2 = (p_ref[0] + p_ref[1]) / jnp.maximum(cnt, 1.0)
    h2 = jnp.maximum(jnp.dot(mean2, w2l_ref[...],
                             preferred_element_type=jnp.float32)
                     + hr_ref[...], 0.0)
    seg = b_ref[0]
    onehot_t = (lax.broadcasted_iota(jnp.int32, (_G, _BR), 0) == seg
                ).astype(jnp.float32)
    acc_ref[...] += jnp.dot(onehot_t, h2, precision=_HI,
                            preferred_element_type=jnp.float32)

    @pl.when(i == _NB - 1)
    def _():
        pooled = acc_ref[...]
        z = jnp.maximum(jnp.dot(pooled, wc1_ref[...],
                                preferred_element_type=jnp.float32)
                        + bc1_ref[...], 0.0)
        z = jnp.maximum(jnp.dot(z, wc2_ref[...],
                                preferred_element_type=jnp.float32)
                        + bc2_ref[...], 0.0)
        out_ref[...] = jnp.dot(z, wc3_ref[...],
                               preferred_element_type=jnp.float32) + bc3_ref[...]


def _tc_post(p2, c, hr2, batch3, w2l, wc1, bc1, wc2, bc2, wc3, bc3):
    return pl.pallas_call(
        _tc_post_body,
        grid=(_NB,),
        in_specs=[
            pl.BlockSpec((_NC, _BR, _D), lambda i: (0, i, 0)),
            pl.BlockSpec((_NC, _BR, _D), lambda i: (0, i, 0)),
            pl.BlockSpec((_BR, _D), lambda i: (i, 0)),
            pl.BlockSpec((1, 1, _BR), lambda i: (i, 0, 0)),
            pl.BlockSpec((_D, _D), lambda i: (0, 0)),
            pl.BlockSpec((_D, _G), lambda i: (0, 0)),
            pl.BlockSpec((1, _G), lambda i: (0, 0)),
            pl.BlockSpec((_G, _G), lambda i: (0, 0)),
            pl.BlockSpec((1, _G), lambda i: (0, 0)),
            pl.BlockSpec((_G, 1), lambda i: (0, 0)),
            pl.BlockSpec((1, 1), lambda i: (0, 0)),
        ],
        out_specs=pl.BlockSpec((_G, 1), lambda i: (0, 0)),
        out_shape=jax.ShapeDtypeStruct((_G, 1), jnp.float32),
        scratch_shapes=[pltpu.VMEM((_G, _D), jnp.float32)],
    )(p2, c, hr2, batch3, w2l, wc1, bc1, wc2, bc2, wc3, bc3)


def _sc_cnt(dst):
    mesh = plsc.VectorSubcoreMesh(core_axis_name="c", subcore_axis_name="s")

    @functools.partial(
        pl.kernel,
        out_type=jax.ShapeDtypeStruct((_NC, _NP, _D), jnp.float32),
        mesh=mesh,
        scratch_types=[
            pltpu.VMEM_SHARED((_NP, _D), jnp.float32),
            pltpu.VMEM((2, _K), jnp.int32),
            pltpu.VMEM((_K, _D), jnp.float32),
            pltpu.VMEM((_ZR, _D), jnp.float32),
            pltpu.SemaphoreType.DMA,
            pltpu.SemaphoreType.DMA,
        ],
    )
    def k(dst_hbm, c_hbm, acc_sh, dstb, ones_v, zrows_v, isem0, isem1):
        c = lax.axis_index("c")
        s = lax.axis_index("s")
        wid = c * _NS + s
        zero16 = jnp.zeros((16,), jnp.float32)
        one16 = jnp.ones((16,), jnp.float32)
        isems = (isem0, isem1)

        def fill(i, carry):
            for j in range(_D // 16):
                zrows_v[i, pl.ds(j * 16, 16)] = zero16
            return carry

        lax.fori_loop(0, _ZR, fill, 0)

        def fill_ones(i, carry):
            for j in range(_D // 16):
                ones_v[i, pl.ds(j * 16, 16)] = one16
            return carry

        lax.fori_loop(0, _K, fill_ones, 0)

        base_r = s * _RB

        def zero_spmem(r, carry):
            pltpu.sync_copy(zrows_v, acc_sh.at[pl.ds(base_r + r * _ZR, _ZR)])
            return carry

        lax.fori_loop(0, _NZ, zero_spmem, 0)
        plsc.subcore_barrier()

        ebase = wid * _EPW

        def issue_idx(chunk_i, b):
            off = ebase + chunk_i * _K
            pltpu.async_copy(dst_hbm.at[pl.ds(off, _K)], dstb.at[b], isems[b])

        def wait_idx(b):
            pltpu.make_async_copy(dst_hbm.at[pl.ds(0, _K)], dstb.at[b],
                                  isems[b]).wait()

        def step(i, b):
            wait_idx(b)
            pltpu.sync_copy(ones_v, acc_sh.at[dstb.at[b]], add=True)
            issue_idx(jnp.minimum(i + 2, _NCHUNK - 1), b)

        issue_idx(0, 0)
        issue_idx(1, 1)

        def outer(j, carry):
            step(2 * j, 0)
            step(2 * j + 1, 1)
            return carry

        lax.fori_loop(0, (_NCHUNK - 1) // 2, outer, 0)
        wait_idx((_NCHUNK - 1) % 2)
        pltpu.sync_copy(ones_v, acc_sh.at[dstb.at[(_NCHUNK - 1) % 2]], add=True)
        wait_idx(_NCHUNK % 2)
        plsc.subcore_barrier()

        pltpu.sync_copy(acc_sh.at[pl.ds(base_r, _RB)],
                        c_hbm.at[c, pl.ds(base_r, _RB)])

    return k(dst)


def _sc_agg(a, src, dst):
    mesh = plsc.VectorSubcoreMesh(core_axis_name="c", subcore_axis_name="s")

    @functools.partial(
        pl.kernel,
        out_type=jax.ShapeDtypeStruct((_NC, _NP, _D), jnp.float32),
        mesh=mesh,
        scratch_types=[
            pltpu.VMEM_SHARED((_NP, _D), jnp.float32),
            pltpu.VMEM((4, _K), jnp.int32),
            pltpu.VMEM((4, _K), jnp.int32),
            pltpu.VMEM((2, _K, _D), jnp.float32),
            pltpu.VMEM((_ZR, _D), jnp.float32),
            pltpu.SemaphoreType.DMA,
            pltpu.SemaphoreType.DMA,
            pltpu.SemaphoreType.DMA,
            pltpu.SemaphoreType.DMA,
            pltpu.SemaphoreType.DMA,
            pltpu.SemaphoreType.DMA,
            pltpu.SemaphoreType.DMA,
            pltpu.SemaphoreType.DMA,
        ],
    )
    def k(a_hbm, src_hbm, dst_hbm, p_hbm, acc_sh, srcb, dstb, rowsb, zrows_v,
          isem0, isem1, isem2, isem3, gsem0, gsem1, ssem0, ssem1):
        c = lax.axis_index("c")
        s = lax.axis_index("s")
        wid = c * _NS + s
        zero16 = jnp.zeros((16,), jnp.float32)
        isems = (isem0, isem1, isem2, isem3)
        gsems = (gsem0, gsem1)
        ssems = (ssem0, ssem1)

        def fill(i, carry):
            for j in range(_D // 16):
                zrows_v[i, pl.ds(j * 16, 16)] = zero16
            return carry

        lax.fori_loop(0, _ZR, fill, 0)

        base_r = s * _RB

        def zero_spmem(r, carry):
            pltpu.sync_copy(zrows_v, acc_sh.at[pl.ds(base_r + r * _ZR, _ZR)])
            return carry

        lax.fori_loop(0, _NZ, zero_spmem, 0)
        plsc.subcore_barrier()

        ebase = wid * _EPW

        def issue_idx(chunk_i, sl):
            off = ebase + chunk_i * _K
            pltpu.async_copy(src_hbm.at[pl.ds(off, _K)], srcb.at[sl], isems[sl])
            pltpu.async_copy(dst_hbm.at[pl.ds(off, _K)], dstb.at[sl], isems[sl])

        def wait_idx(sl):
            pltpu.make_async_copy(src_hbm.at[pl.ds(0, _K)], srcb.at[sl],
                                  isems[sl]).wait()
            pltpu.make_async_copy(dst_hbm.at[pl.ds(0, _K)], dstb.at[sl],
                                  isems[sl]).wait()

        def issue_gather(sl, rb):
            pltpu.async_copy(a_hbm.at[srcb.at[sl]], rowsb.at[rb], gsems[rb])

        def wait_gather(sl, rb):
            pltpu.make_async_copy(a_hbm.at[srcb.at[sl]], rowsb.at[rb],
                                  gsems[rb]).wait()

        def issue_scatter(sl, rb):
            pltpu.async_copy(rowsb.at[rb], acc_sh.at[dstb.at[sl]],
                             ssems[rb], add=True)

        def wait_scatter(sl, rb):
            pltpu.make_async_copy(rowsb.at[rb], acc_sh.at[dstb.at[sl]],
                                  ssems[rb]).wait()

        issue_idx(0, 0)
        issue_idx(1, 1)
        issue_idx(2, 2)
        wait_idx(0)
        issue_gather(0, 0)

        wait_idx(1)
        issue_gather(1, 1)
        issue_idx(3, 3)
        wait_gather(0, 0)
        issue_scatter(0, 0)

        wait_idx(2)
        wait_scatter(0, 0)
        issue_gather(2, 0)
        issue_idx(4, 0)
        wait_gather(1, 1)
        issue_scatter(1, 1)

        def step(i, sl, rb):
            sl1 = (sl + 1) % 4
            sl3 = (sl + 3) % 4
            slm = (sl - 1) % 4
            rb1 = 1 - rb
            wait_idx(sl1)
            wait_scatter(slm, rb1)
            issue_gather(sl1, rb1)
            issue_idx(jnp.minimum(i + 3, _NCHUNK - 1), sl3)
            wait_gather(sl, rb)
            issue_scatter(sl, rb)

        def outer(j, carry):
            i = 4 * j + 2
            step(i, 2, 0)
            step(i + 1, 3, 1)
            step(i + 2, 0, 0)
            step(i + 3, 1, 1)
            return carry

        lax.fori_loop(0, 30, outer, 0)

        wait_idx(3)
        wait_scatter(1, 1)
        issue_gather(3, 1)
        issue_idx(_NCHUNK - 1, 1)
        wait_gather(2, 0)
        issue_scatter(2, 0)

        wait_idx(0)
        wait_scatter(2, 0)
        issue_gather(0, 0)
        issue_idx(_NCHUNK - 1, 2)
        wait_gather(3, 1)
        issue_scatter(3, 1)

        wait_scatter(3, 1)
        wait_gather(0, 0)
        issue_scatter(0, 0)

        wait_scatter(0, 0)
        wait_idx(1)
        wait_idx(2)
        plsc.subcore_barrier()

        pltpu.sync_copy(acc_sh.at[pl.ds(base_r, _RB)],
                        p_hbm.at[c, pl.ds(base_r, _RB)])

    return k(a, src, dst)


def kernel(x, edge_index, batch, W1l, W1r, b1, gamma, beta, W2l, W2r, b2,
           Wc1, bc1, Wc2, bc2, Wc3, bc3):
    src = edge_index[0]
    dst = edge_index[1]
    xp = jnp.pad(x, ((0, _NP - _N), (0, 0)))
    batchp = jnp.pad(batch, (0, _NP - _N), constant_values=_G)

    cpart = _sc_cnt(dst)
    p1 = _sc_agg(xp, src, dst)
    h, hr2 = _tc_mid(p1, cpart, xp, gamma.reshape(1, _D), beta.reshape(1, _D),
                     W1l, W1r, b1.reshape(1, _D), W2r, b2.reshape(1, _D))
    p2 = _sc_agg(h, src, dst)
    pred = _tc_post(p2, cpart, hr2, batchp.reshape(_NB, 1, _BR),
                    W2l, Wc1, bc1.reshape(1, _G), Wc2, bc2.reshape(1, _G),
                    Wc3, bc3.reshape(1, 1))
    return pred


# speedup vs baseline: 9.9207x; 1.0100x over previous
import functools

import jax
import jax.numpy as jnp
from jax import lax
from jax.experimental import pallas as pl
from jax.experimental.pallas import tpu as pltpu
from jax.experimental.pallas import tpu_sc as plsc

_N = 10000
_E = 320000
_D = 128
_G = 64

_NP = 10240
_NB = 10
_BR = _NP // _NB
_NC = 2
_NS = 16
_NW = _NC * _NS
_EPW = _E // _NW
_K = 80
_NCHUNK = _EPW // _K
_RB = _NP // _NS
_ZR = 64
_NZ = _RB // _ZR

_HI = jax.lax.Precision.HIGHEST


def _tc_mid_body(p_ref, c_ref, x_ref, g_ref, bt_ref, w1l_ref, w1r_ref, b1_ref,
                 w2r_ref, b2_ref, h_ref, hr2_ref):
    cnt = jnp.sum(c_ref[0] + c_ref[1], axis=1, keepdims=True) * (1.0 / _D)
    mean1 = (p_ref[0] + p_ref[1]) / jnp.maximum(cnt, 1.0)
    h = jnp.maximum(jnp.dot(mean1, w1l_ref[...],
                            preferred_element_type=jnp.float32)
                    + jnp.dot(x_ref[...], w1r_ref[...],
                              preferred_element_type=jnp.float32)
                    + b1_ref[...], 0.0)
    mu = jnp.mean(h, axis=1, keepdims=True)
    d = h - mu
    var = jnp.mean(d * d, axis=1, keepdims=True)
    h = d / jnp.sqrt(var + 1e-5) * g_ref[...] + bt_ref[...]
    h_ref[...] = h
    hr2_ref[...] = jnp.dot(h, w2r_ref[...],
                           preferred_element_type=jnp.float32) + b2_ref[...]


def _tc_mid(p1, c, xp, gamma, beta, w1l, w1r, b1, w2r, b2):
    return pl.pallas_call(
        _tc_mid_body,
        grid=(_NB,),
        in_specs=[
            pl.BlockSpec((_NC, _BR, _D), lambda i: (0, i, 0)),
            pl.BlockSpec((_NC, _BR, _D), lambda i: (0, i, 0)),
            pl.BlockSpec((_BR, _D), lambda i: (i, 0)),
            pl.BlockSpec((1, _D), lambda i: (0, 0)),
            pl.BlockSpec((1, _D), lambda i: (0, 0)),
            pl.BlockSpec((_D, _D), lambda i: (0, 0)),
            pl.BlockSpec((_D, _D), lambda i: (0, 0)),
            pl.BlockSpec((1, _D), lambda i: (0, 0)),
            pl.BlockSpec((_D, _D), lambda i: (0, 0)),
            pl.BlockSpec((1, _D), lambda i: (0, 0)),
        ],
        out_specs=[
            pl.BlockSpec((_BR, _D), lambda i: (i, 0)),
            pl.BlockSpec((_BR, _D), lambda i: (i, 0)),
        ],
        out_shape=[
            jax.ShapeDtypeStruct((_NP, _D), jnp.float32),
            jax.ShapeDtypeStruct((_NP, _D), jnp.float32),
        ],
    )(p1, c, xp, gamma, beta, w1l, w1r, b1, w2r, b2)


def _tc_post_body(p_ref, c_ref, hr_ref, b_ref, w2l_ref, wc1_ref, bc1_ref,
                  wc2_ref, bc2_ref, wc3_ref, bc3_ref, out_ref, acc_ref):
    i = pl.program_id(0)

    @pl.when(i == 0)
    def _():
        acc_ref[...] = jnp.zeros_like(acc_ref)

    cnt = jnp.sum(c_ref[0] + c_ref[1], axis=1, keepdims=True) * (1.0 / _D)
    mean2 = (p_ref[0] + p_ref[1]) / jnp.maximum(cnt, 1.0)
    h2 = jnp.maximum(jnp.dot(mean2, w2l_ref[...],
                             preferred_element_type=jnp.float32)
                     + hr_ref[...], 0.0)
    seg = b_ref[0]
    onehot_t = (lax.broadcasted_iota(jnp.int32, (_G, _BR), 0) == seg
                ).astype(jnp.float32)
    acc_ref[...] += jnp.dot(onehot_t, h2, precision=_HI,
                            preferred_element_type=jnp.float32)

    @pl.when(i == _NB - 1)
    def _():
        pooled = acc_ref[...]
        z = jnp.maximum(jnp.dot(pooled, wc1_ref[...],
                                preferred_element_type=jnp.float32)
                        + bc1_ref[...], 0.0)
        z = jnp.maximum(jnp.dot(z, wc2_ref[...],
                                preferred_element_type=jnp.float32)
                        + bc2_ref[...], 0.0)
        out_ref[...] = jnp.dot(z, wc3_ref[...],
                               preferred_element_type=jnp.float32) + bc3_ref[...]


def _tc_post(p2, c, hr2, batch3, w2l, wc1, bc1, wc2, bc2, wc3, bc3):
    return pl.pallas_call(
        _tc_post_body,
        grid=(_NB,),
        in_specs=[
            pl.BlockSpec((_NC, _BR, _D), lambda i: (0, i, 0)),
            pl.BlockSpec((_NC, _BR, _D), lambda i: (0, i, 0)),
            pl.BlockSpec((_BR, _D), lambda i: (i, 0)),
            pl.BlockSpec((1, 1, _BR), lambda i: (i, 0, 0)),
            pl.BlockSpec((_D, _D), lambda i: (0, 0)),
            pl.BlockSpec((_D, _G), lambda i: (0, 0)),
            pl.BlockSpec((1, _G), lambda i: (0, 0)),
            pl.BlockSpec((_G, _G), lambda i: (0, 0)),
            pl.BlockSpec((1, _G), lambda i: (0, 0)),
            pl.BlockSpec((_G, 1), lambda i: (0, 0)),
            pl.BlockSpec((1, 1), lambda i: (0, 0)),
        ],
        out_specs=pl.BlockSpec((_G, 1), lambda i: (0, 0)),
        out_shape=jax.ShapeDtypeStruct((_G, 1), jnp.float32),
        scratch_shapes=[pltpu.VMEM((_G, _D), jnp.float32)],
    )(p2, c, hr2, batch3, w2l, wc1, bc1, wc2, bc2, wc3, bc3)


def _sc_cnt(dst):
    mesh = plsc.VectorSubcoreMesh(core_axis_name="c", subcore_axis_name="s")

    @functools.partial(
        pl.kernel,
        out_type=jax.ShapeDtypeStruct((_NC, _NP, _D), jnp.float32),
        mesh=mesh,
        scratch_types=[
            pltpu.VMEM_SHARED((_NP, _D), jnp.float32),
            pltpu.VMEM((4, _K), jnp.int32),
            pltpu.VMEM((_K, _D), jnp.float32),
            pltpu.VMEM((_ZR, _D), jnp.float32),
            pltpu.SemaphoreType.DMA,
            pltpu.SemaphoreType.DMA,
            pltpu.SemaphoreType.DMA,
            pltpu.SemaphoreType.DMA,
            pltpu.SemaphoreType.DMA,
            pltpu.SemaphoreType.DMA,
        ],
    )
    def k(dst_hbm, c_hbm, acc_sh, dstb, ones_v, zrows_v,
          isem0, isem1, isem2, isem3, ssem0, ssem1):
        c = lax.axis_index("c")
        s = lax.axis_index("s")
        wid = c * _NS + s
        zero16 = jnp.zeros((16,), jnp.float32)
        one16 = jnp.ones((16,), jnp.float32)
        isems = (isem0, isem1, isem2, isem3)
        ssems = (ssem0, ssem1)
        ebase = wid * _EPW

        def issue_idx(chunk_i, sl):
            off = ebase + chunk_i * _K
            pltpu.async_copy(dst_hbm.at[pl.ds(off, _K)], dstb.at[sl], isems[sl])

        def wait_idx(sl):
            pltpu.make_async_copy(dst_hbm.at[pl.ds(0, _K)], dstb.at[sl],
                                  isems[sl]).wait()

        def issue_scatter(sl, sm):
            pltpu.async_copy(ones_v, acc_sh.at[dstb.at[sl]], ssems[sm],
                             add=True)

        def wait_scatter(sl, sm):
            pltpu.make_async_copy(ones_v, acc_sh.at[dstb.at[sl]],
                                  ssems[sm]).wait()

        issue_idx(0, 0)
        issue_idx(1, 1)

        def fill(i, carry):
            for j in range(_D // 16):
                zrows_v[i, pl.ds(j * 16, 16)] = zero16
                ones_v[i, pl.ds(j * 16, 16)] = one16
            return carry

        lax.fori_loop(0, _ZR, fill, 0)

        def fill_ones(i, carry):
            for j in range(_D // 16):
                ones_v[i, pl.ds(j * 16, 16)] = one16
            return carry

        lax.fori_loop(_ZR, _K, fill_ones, 0)

        base_r = s * _RB

        def zero_spmem(r, carry):
            pltpu.sync_copy(zrows_v, acc_sh.at[pl.ds(base_r + r * _ZR, _ZR)])
            return carry

        lax.fori_loop(0, _NZ, zero_spmem, 0)
        plsc.subcore_barrier()

        issue_idx(2, 2)
        wait_idx(0)
        issue_scatter(0, 0)
        issue_idx(3, 3)
        wait_idx(1)
        issue_scatter(1, 1)

        def step(i, sl, sm):
            sl2 = (sl + 2) % 4
            wait_scatter(sl2, sm)
            issue_idx(jnp.minimum(i + 2, _NCHUNK - 1), sl2)
            wait_idx(sl)
            issue_scatter(sl, sm)

        def outer(j, carry):
            i = 4 * j + 2
            step(i, 2, 0)
            step(i + 1, 3, 1)
            step(i + 2, 0, 0)
            step(i + 3, 1, 1)
            return carry

        lax.fori_loop(0, 30, outer, 0)

        wait_scatter(0, 0)
        issue_idx(_NCHUNK - 1, 0)
        wait_idx(2)
        issue_scatter(2, 0)
        wait_scatter(1, 1)
        wait_idx(3)
        issue_scatter(3, 1)
        wait_scatter(2, 0)
        wait_idx(0)
        issue_scatter(0, 0)

        wait_scatter(3, 1)
        wait_scatter(0, 0)
        plsc.subcore_barrier()

        pltpu.sync_copy(acc_sh.at[pl.ds(base_r, _RB)],
                        c_hbm.at[c, pl.ds(base_r, _RB)])

    return k(dst)


def _sc_agg(a, src, dst):
    mesh = plsc.VectorSubcoreMesh(core_axis_name="c", subcore_axis_name="s")

    @functools.partial(
        pl.kernel,
        out_type=jax.ShapeDtypeStruct((_NC, _NP, _D), jnp.float32),
        mesh=mesh,
        scratch_types=[
            pltpu.VMEM_SHARED((_NP, _D), jnp.float32),
            pltpu.VMEM((4, _K), jnp.int32),
            pltpu.VMEM((4, _K), jnp.int32),
            pltpu.VMEM((2, _K, _D), jnp.float32),
            pltpu.VMEM((_ZR, _D), jnp.float32),
            pltpu.SemaphoreType.DMA,
            pltpu.SemaphoreType.DMA,
            pltpu.SemaphoreType.DMA,
            pltpu.SemaphoreType.DMA,
            pltpu.SemaphoreType.DMA,
            pltpu.SemaphoreType.DMA,
            pltpu.SemaphoreType.DMA,
            pltpu.SemaphoreType.DMA,
        ],
    )
    def k(a_hbm, src_hbm, dst_hbm, p_hbm, acc_sh, srcb, dstb, rowsb, zrows_v,
          isem0, isem1, isem2, isem3, gsem0, gsem1, ssem0, ssem1):
        c = lax.axis_index("c")
        s = lax.axis_index("s")
        wid = c * _NS + s
        zero16 = jnp.zeros((16,), jnp.float32)
        isems = (isem0, isem1, isem2, isem3)
        gsems = (gsem0, gsem1)
        ssems = (ssem0, ssem1)

        ebase0 = wid * _EPW
        pltpu.async_copy(src_hbm.at[pl.ds(ebase0, _K)], srcb.at[0], isems[0])
        pltpu.async_copy(dst_hbm.at[pl.ds(ebase0, _K)], dstb.at[0], isems[0])
        pltpu.async_copy(src_hbm.at[pl.ds(ebase0 + _K, _K)], srcb.at[1],
                         isems[1])
        pltpu.async_copy(dst_hbm.at[pl.ds(ebase0 + _K, _K)], dstb.at[1],
                         isems[1])

        def fill(i, carry):
            for j in range(_D // 16):
                zrows_v[i, pl.ds(j * 16, 16)] = zero16
            return carry

        lax.fori_loop(0, _ZR, fill, 0)

        base_r = s * _RB

        def zero_spmem(r, carry):
            pltpu.sync_copy(zrows_v, acc_sh.at[pl.ds(base_r + r * _ZR, _ZR)])
            return carry

        lax.fori_loop(0, _NZ, zero_spmem, 0)
        plsc.subcore_barrier()

        ebase = wid * _EPW

        def issue_idx(chunk_i, sl):
            off = ebase + chunk_i * _K
            pltpu.async_copy(src_hbm.at[pl.ds(off, _K)], srcb.at[sl], isems[sl])
            pltpu.async_copy(dst_hbm.at[pl.ds(off, _K)], dstb.at[sl], isems[sl])

        def wait_idx(sl):
            pltpu.make_async_copy(src_hbm.at[pl.ds(0, _K)], srcb.at[sl],
                                  isems[sl]).wait()
            pltpu.make_async_copy(dst_hbm.at[pl.ds(0, _K)], dstb.at[sl],
                                  isems[sl]).wait()

        def issue_gather(sl, rb):
            pltpu.async_copy(a_hbm.at[srcb.at[sl]], rowsb.at[rb], gsems[rb])

        def wait_gather(sl, rb):
            pltpu.make_async_copy(a_hbm.at[srcb.at[sl]], rowsb.at[rb],
                                  gsems[rb]).wait()

        def issue_scatter(sl, rb):
            pltpu.async_copy(rowsb.at[rb], acc_sh.at[dstb.at[sl]],
                             ssems[rb], add=True)

        def wait_scatter(sl, rb):
            pltpu.make_async_copy(rowsb.at[rb], acc_sh.at[dstb.at[sl]],
                                  ssems[rb]).wait()

        issue_idx(2, 2)
        wait_idx(0)
        issue_gather(0, 0)

        wait_idx(1)
        issue_gather(1, 1)
        issue_idx(3, 3)
        wait_gather(0, 0)
        issue_scatter(0, 0)

        wait_idx(2)
        wait_scatter(0, 0)
        issue_gather(2, 0)
        issue_idx(4, 0)
        wait_gather(1, 1)
        issue_scatter(1, 1)

        def step(i, sl, rb):
            sl1 = (sl + 1) % 4
            sl3 = (sl + 3) % 4
            slm = (sl - 1) % 4
            rb1 = 1 - rb
            wait_idx(sl1)
            wait_scatter(slm, rb1)
            issue_gather(sl1, rb1)
            issue_idx(jnp.minimum(i + 3, _NCHUNK - 1), sl3)
            wait_gather(sl, rb)
            issue_scatter(sl, rb)

        def outer(j, carry):
            i = 4 * j + 2
            step(i, 2, 0)
            step(i + 1, 3, 1)
            step(i + 2, 0, 0)
            step(i + 3, 1, 1)
            return carry

        lax.fori_loop(0, 30, outer, 0)

        wait_idx(3)
        wait_scatter(1, 1)
        issue_gather(3, 1)
        issue_idx(_NCHUNK - 1, 1)
        wait_gather(2, 0)
        issue_scatter(2, 0)

        wait_idx(0)
        wait_scatter(2, 0)
        issue_gather(0, 0)
        issue_idx(_NCHUNK - 1, 2)
        wait_gather(3, 1)
        issue_scatter(3, 1)

        wait_scatter(3, 1)
        wait_gather(0, 0)
        issue_scatter(0, 0)

        wait_scatter(0, 0)
        wait_idx(1)
        wait_idx(2)
        plsc.subcore_barrier()

        pltpu.sync_copy(acc_sh.at[pl.ds(base_r, _RB)],
                        p_hbm.at[c, pl.ds(base_r, _RB)])

    return k(a, src, dst)


def kernel(x, edge_index, batch, W1l, W1r, b1, gamma, beta, W2l, W2r, b2,
           Wc1, bc1, Wc2, bc2, Wc3, bc3):
    src = edge_index[0]
    dst = edge_index[1]
    xp = jnp.pad(x, ((0, _NP - _N), (0, 0)))
    batchp = jnp.pad(batch, (0, _NP - _N), constant_values=_G)

    cpart = _sc_cnt(dst)
    p1 = _sc_agg(xp, src, dst)
    h, hr2 = _tc_mid(p1, cpart, xp, gamma.reshape(1, _D), beta.reshape(1, _D),
                     W1l, W1r, b1.reshape(1, _D), W2r, b2.reshape(1, _D))
    p2 = _sc_agg(h, src, dst)
    pred = _tc_post(p2, cpart, hr2, batchp.reshape(_NB, 1, _BR),
                    W2l, Wc1, bc1.reshape(1, _G), Wc2, bc2.reshape(1, _G),
                    Wc3, bc3.reshape(1, 1))
    return pred


# speedup vs baseline: 11.0250x; 1.1113x over previous
import functools

import jax
import jax.numpy as jnp
from jax import lax
from jax.experimental import pallas as pl
from jax.experimental.pallas import tpu as pltpu
from jax.experimental.pallas import tpu_sc as plsc

_N = 10000
_E = 320000
_D = 128
_G = 64

_NP = 10240
_NB = 10
_BR = _NP // _NB
_NC = 2
_NS = 16
_NW = _NC * _NS
_EPW = _E // _NW
_K = 80
_NCHUNK = _EPW // _K
_RB = _NP // _NS
_ZR = 64
_NZ = _RB // _ZR

_HI = jax.lax.Precision.HIGHEST


def _tc_mid_body(p_ref, c_ref, x_ref, g_ref, bt_ref, w1l_ref, w1r_ref, b1_ref,
                 w2r_ref, b2_ref, h_ref, hr2_ref):
    cnt = jnp.sum(c_ref[0] + c_ref[1], axis=1, keepdims=True) * (1.0 / _D)
    mean1 = (p_ref[0] + p_ref[1]) / jnp.maximum(cnt, 1.0)
    h = jnp.maximum(jnp.dot(mean1, w1l_ref[...],
                            preferred_element_type=jnp.float32)
                    + jnp.dot(x_ref[...], w1r_ref[...],
                              preferred_element_type=jnp.float32)
                    + b1_ref[...], 0.0)
    mu = jnp.mean(h, axis=1, keepdims=True)
    d = h - mu
    var = jnp.mean(d * d, axis=1, keepdims=True)
    h = d / jnp.sqrt(var + 1e-5) * g_ref[...] + bt_ref[...]
    h_ref[...] = h
    hr2_ref[...] = jnp.dot(h, w2r_ref[...],
                           preferred_element_type=jnp.float32) + b2_ref[...]


def _tc_mid(p1, c, xp, gamma, beta, w1l, w1r, b1, w2r, b2):
    return pl.pallas_call(
        _tc_mid_body,
        grid=(_NB,),
        in_specs=[
            pl.BlockSpec((_NC, _BR, _D), lambda i: (0, i, 0)),
            pl.BlockSpec((_NC, _BR, _D), lambda i: (0, i, 0)),
            pl.BlockSpec((_BR, _D), lambda i: (i, 0)),
            pl.BlockSpec((1, _D), lambda i: (0, 0)),
            pl.BlockSpec((1, _D), lambda i: (0, 0)),
            pl.BlockSpec((_D, _D), lambda i: (0, 0)),
            pl.BlockSpec((_D, _D), lambda i: (0, 0)),
            pl.BlockSpec((1, _D), lambda i: (0, 0)),
            pl.BlockSpec((_D, _D), lambda i: (0, 0)),
            pl.BlockSpec((1, _D), lambda i: (0, 0)),
        ],
        out_specs=[
            pl.BlockSpec((_BR, _D), lambda i: (i, 0)),
            pl.BlockSpec((_BR, _D), lambda i: (i, 0)),
        ],
        out_shape=[
            jax.ShapeDtypeStruct((_NP, _D), jnp.float32),
            jax.ShapeDtypeStruct((_NP, _D), jnp.float32),
        ],
    )(p1, c, xp, gamma, beta, w1l, w1r, b1, w2r, b2)


def _tc_post_body(p_ref, c_ref, hr_ref, b_ref, w2l_ref, wc1_ref, bc1_ref,
                  wc2_ref, bc2_ref, wc3_ref, bc3_ref, out_ref, acc_ref):
    i = pl.program_id(0)

    @pl.when(i == 0)
    def _():
        acc_ref[...] = jnp.zeros_like(acc_ref)

    cnt = jnp.sum(c_ref[0] + c_ref[1], axis=1, keepdims=True) * (1.0 / _D)
    mean2 = (p_ref[0] + p_ref[1]) / jnp.maximum(cnt, 1.0)
    h2 = jnp.maximum(jnp.dot(mean2, w2l_ref[...],
                             preferred_element_type=jnp.float32)
                     + hr_ref[...], 0.0)
    seg = b_ref[0]
    onehot_t = (lax.broadcasted_iota(jnp.int32, (_G, _BR), 0) == seg
                ).astype(jnp.float32)
    acc_ref[...] += jnp.dot(onehot_t, h2, precision=_HI,
                            preferred_element_type=jnp.float32)

    @pl.when(i == _NB - 1)
    def _():
        pooled = acc_ref[...]
        z = jnp.maximum(jnp.dot(pooled, wc1_ref[...],
                                preferred_element_type=jnp.float32)
                        + bc1_ref[...], 0.0)
        z = jnp.maximum(jnp.dot(z, wc2_ref[...],
                                preferred_element_type=jnp.float32)
                        + bc2_ref[...], 0.0)
        out_ref[...] = jnp.dot(z, wc3_ref[...],
                               preferred_element_type=jnp.float32) + bc3_ref[...]


def _tc_post(p2, c, hr2, batch3, w2l, wc1, bc1, wc2, bc2, wc3, bc3):
    return pl.pallas_call(
        _tc_post_body,
        grid=(_NB,),
        in_specs=[
            pl.BlockSpec((_NC, _BR, _D), lambda i: (0, i, 0)),
            pl.BlockSpec((_NC, _BR, _D), lambda i: (0, i, 0)),
            pl.BlockSpec((_BR, _D), lambda i: (i, 0)),
            pl.BlockSpec((1, 1, _BR), lambda i: (i, 0, 0)),
            pl.BlockSpec((_D, _D), lambda i: (0, 0)),
            pl.BlockSpec((_D, _G), lambda i: (0, 0)),
            pl.BlockSpec((1, _G), lambda i: (0, 0)),
            pl.BlockSpec((_G, _G), lambda i: (0, 0)),
            pl.BlockSpec((1, _G), lambda i: (0, 0)),
            pl.BlockSpec((_G, 1), lambda i: (0, 0)),
            pl.BlockSpec((1, 1), lambda i: (0, 0)),
        ],
        out_specs=pl.BlockSpec((_G, 1), lambda i: (0, 0)),
        out_shape=jax.ShapeDtypeStruct((_G, 1), jnp.float32),
        scratch_shapes=[pltpu.VMEM((_G, _D), jnp.float32)],
    )(p2, c, hr2, batch3, w2l, wc1, bc1, wc2, bc2, wc3, bc3)


def _sc_cnt(dst):
    mesh = plsc.VectorSubcoreMesh(core_axis_name="c", subcore_axis_name="s")

    @functools.partial(
        pl.kernel,
        out_type=jax.ShapeDtypeStruct((_NC, _NP, _D), jnp.float32),
        mesh=mesh,
        scratch_types=[
            pltpu.VMEM_SHARED((_NP, _D), jnp.float32),
            pltpu.VMEM((4, _K), jnp.int32),
            pltpu.VMEM((_K, _D), jnp.float32),
            pltpu.VMEM((_ZR, _D), jnp.float32),
            pltpu.SemaphoreType.DMA,
            pltpu.SemaphoreType.DMA,
            pltpu.SemaphoreType.DMA,
            pltpu.SemaphoreType.DMA,
            pltpu.SemaphoreType.DMA,
            pltpu.SemaphoreType.DMA,
        ],
    )
    def k(dst_hbm, c_hbm, acc_sh, dstb, ones_v, zrows_v,
          isem0, isem1, isem2, isem3, ssem0, ssem1):
        c = lax.axis_index("c")
        s = lax.axis_index("s")
        wid = c * _NS + s
        zero16 = jnp.zeros((16,), jnp.float32)
        one16 = jnp.ones((16,), jnp.float32)
        isems = (isem0, isem1, isem2, isem3)
        ssems = (ssem0, ssem1)
        ebase = wid * _EPW

        def issue_idx(chunk_i, sl):
            off = ebase + chunk_i * _K
            pltpu.async_copy(dst_hbm.at[pl.ds(off, _K)], dstb.at[sl], isems[sl])

        def wait_idx(sl):
            pltpu.make_async_copy(dst_hbm.at[pl.ds(0, _K)], dstb.at[sl],
                                  isems[sl]).wait()

        def issue_scatter(sl, sm):
            pltpu.async_copy(ones_v, acc_sh.at[dstb.at[sl]], ssems[sm],
                             add=True)

        def wait_scatter(sl, sm):
            pltpu.make_async_copy(ones_v, acc_sh.at[dstb.at[sl]],
                                  ssems[sm]).wait()

        issue_idx(0, 0)
        issue_idx(1, 1)

        def fill(i, carry):
            for j in range(_D // 16):
                zrows_v[i, pl.ds(j * 16, 16)] = zero16
                ones_v[i, pl.ds(j * 16, 16)] = one16
            return carry

        lax.fori_loop(0, _ZR, fill, 0)

        def fill_ones(i, carry):
            for j in range(_D // 16):
                ones_v[i, pl.ds(j * 16, 16)] = one16
            return carry

        lax.fori_loop(_ZR, _K, fill_ones, 0)

        base_r = s * _RB

        def zero_spmem(r, carry):
            pltpu.sync_copy(zrows_v, acc_sh.at[pl.ds(base_r + r * _ZR, _ZR)])
            return carry

        lax.fori_loop(0, _NZ, zero_spmem, 0)
        plsc.subcore_barrier()

        issue_idx(2, 2)
        wait_idx(0)
        issue_scatter(0, 0)
        issue_idx(3, 3)
        wait_idx(1)
        issue_scatter(1, 1)

        def step(i, sl, sm):
            sl2 = (sl + 2) % 4
            wait_scatter(sl2, sm)
            issue_idx(jnp.minimum(i + 2, _NCHUNK - 1), sl2)
            wait_idx(sl)
            issue_scatter(sl, sm)

        def outer(j, carry):
            i = 4 * j + 2
            step(i, 2, 0)
            step(i + 1, 3, 1)
            step(i + 2, 0, 0)
            step(i + 3, 1, 1)
            return carry

        lax.fori_loop(0, 30, outer, 0)

        wait_scatter(0, 0)
        issue_idx(_NCHUNK - 1, 0)
        wait_idx(2)
        issue_scatter(2, 0)
        wait_scatter(1, 1)
        wait_idx(3)
        issue_scatter(3, 1)
        wait_scatter(2, 0)
        wait_idx(0)
        issue_scatter(0, 0)

        wait_scatter(3, 1)
        wait_scatter(0, 0)
        plsc.subcore_barrier()

        pltpu.sync_copy(acc_sh.at[pl.ds(base_r, _RB)],
                        c_hbm.at[c, pl.ds(base_r, _RB)])

    return k(dst)


def _sc_agg(a, src, dst):
    mesh = plsc.VectorSubcoreMesh(core_axis_name="c", subcore_axis_name="s")

    @functools.partial(
        pl.kernel,
        out_type=jax.ShapeDtypeStruct((_NC, _NP, _D), jnp.float32),
        mesh=mesh,
        scratch_types=[
            pltpu.VMEM_SHARED((_NP, _D), jnp.float32),
            pltpu.VMEM((6, _K), jnp.int32),
            pltpu.VMEM((6, _K), jnp.int32),
            pltpu.VMEM((3, _K, _D), jnp.float32),
            pltpu.VMEM((_ZR, _D), jnp.float32),
            pltpu.SemaphoreType.DMA,
            pltpu.SemaphoreType.DMA,
            pltpu.SemaphoreType.DMA,
            pltpu.SemaphoreType.DMA,
            pltpu.SemaphoreType.DMA,
            pltpu.SemaphoreType.DMA,
            pltpu.SemaphoreType.DMA,
            pltpu.SemaphoreType.DMA,
            pltpu.SemaphoreType.DMA,
            pltpu.SemaphoreType.DMA,
            pltpu.SemaphoreType.DMA,
            pltpu.SemaphoreType.DMA,
        ],
    )
    def k(a_hbm, src_hbm, dst_hbm, p_hbm, acc_sh, srcb, dstb, rowsb, zrows_v,
          isem0, isem1, isem2, isem3, isem4, isem5,
          gsem0, gsem1, gsem2, ssem0, ssem1, ssem2):
        c = lax.axis_index("c")
        s = lax.axis_index("s")
        wid = c * _NS + s
        zero16 = jnp.zeros((16,), jnp.float32)
        isems = (isem0, isem1, isem2, isem3, isem4, isem5)
        gsems = (gsem0, gsem1, gsem2)
        ssems = (ssem0, ssem1, ssem2)
        ebase = wid * _EPW

        def issue_idx(chunk_i, sl):
            off = ebase + chunk_i * _K
            pltpu.async_copy(src_hbm.at[pl.ds(off, _K)], srcb.at[sl], isems[sl])
            pltpu.async_copy(dst_hbm.at[pl.ds(off, _K)], dstb.at[sl], isems[sl])

        def wait_idx(sl):
            pltpu.make_async_copy(src_hbm.at[pl.ds(0, _K)], srcb.at[sl],
                                  isems[sl]).wait()
            pltpu.make_async_copy(dst_hbm.at[pl.ds(0, _K)], dstb.at[sl],
                                  isems[sl]).wait()

        def issue_gather(sl, rb):
            pltpu.async_copy(a_hbm.at[srcb.at[sl]], rowsb.at[rb], gsems[rb])

        def wait_gather(sl, rb):
            pltpu.make_async_copy(a_hbm.at[srcb.at[sl]], rowsb.at[rb],
                                  gsems[rb]).wait()

        def issue_scatter(sl, rb, sm):
            pltpu.async_copy(rowsb.at[rb], acc_sh.at[dstb.at[sl]],
                             ssems[sm], add=True)

        def wait_scatter(sl, rb, sm):
            pltpu.make_async_copy(rowsb.at[rb], acc_sh.at[dstb.at[sl]],
                                  ssems[sm]).wait()

        issue_idx(0, 0)
        issue_idx(1, 1)
        issue_idx(2, 2)
        issue_idx(3, 3)

        def fill(i, carry):
            for j in range(_D // 16):
                zrows_v[i, pl.ds(j * 16, 16)] = zero16
            return carry

        lax.fori_loop(0, _ZR, fill, 0)

        base_r = s * _RB

        def zero_spmem(r, carry):
            pltpu.sync_copy(zrows_v, acc_sh.at[pl.ds(base_r + r * _ZR, _ZR)])
            return carry

        lax.fori_loop(0, _NZ, zero_spmem, 0)
        plsc.subcore_barrier()

        issue_idx(4, 4)
        wait_idx(0)
        issue_gather(0, 0)
        wait_idx(1)
        issue_gather(1, 1)
        wait_gather(0, 0)
        issue_scatter(0, 0, 0)
        issue_idx(5, 5)
        wait_idx(2)
        issue_gather(2, 2)
        wait_gather(1, 1)
        issue_scatter(1, 1, 1)

        def step(i, sl, rb, sm):
            slm2 = (sl + 4) % 6
            rbm2 = (rb + 1) % 3
            sl1 = (sl + 1) % 6
            wait_scatter(slm2, rbm2, rbm2)
            issue_idx(jnp.minimum(i + 4, _NCHUNK - 1), slm2)
            wait_idx(sl1)
            issue_gather(sl1, rbm2)
            wait_gather(sl, rb)
            issue_scatter(sl, rb, sm)

        def outer(j, carry):
            i = 6 * j + 2
            step(i, 2, 2, 2)
            step(i + 1, 3, 0, 0)
            step(i + 2, 4, 1, 1)
            step(i + 3, 5, 2, 2)
            step(i + 4, 0, 0, 0)
            step(i + 5, 1, 1, 1)
            return carry

        lax.fori_loop(0, 20, outer, 0)

        wait_scatter(0, 0, 0)
        issue_idx(_NCHUNK - 1, 0)
        wait_idx(3)
        issue_gather(3, 0)
        wait_gather(2, 2)
        issue_scatter(2, 2, 2)
        wait_scatter(1, 1, 1)
        issue_idx(_NCHUNK - 1, 1)
        wait_idx(4)
        issue_gather(4, 1)
        wait_gather(3, 0)
        issue_scatter(3, 0, 0)
        wait_scatter(2, 2, 2)
        wait_gather(4, 1)
        issue_scatter(4, 1, 1)

        wait_scatter(3, 0, 0)
        wait_scatter(4, 1, 1)
        wait_idx(5)
        wait_idx(0)
        wait_idx(1)
        plsc.subcore_barrier()

        pltpu.sync_copy(acc_sh.at[pl.ds(base_r, _RB)],
                        p_hbm.at[c, pl.ds(base_r, _RB)])

    return k(a, src, dst)


def kernel(x, edge_index, batch, W1l, W1r, b1, gamma, beta, W2l, W2r, b2,
           Wc1, bc1, Wc2, bc2, Wc3, bc3):
    src = edge_index[0]
    dst = edge_index[1]
    xp = jnp.pad(x, ((0, _NP - _N), (0, 0)))
    batchp = jnp.pad(batch, (0, _NP - _N), constant_values=_G)

    cpart = _sc_cnt(dst)
    p1 = _sc_agg(xp, src, dst)
    h, hr2 = _tc_mid(p1, cpart, xp, gamma.reshape(1, _D), beta.reshape(1, _D),
                     W1l, W1r, b1.reshape(1, _D), W2r, b2.reshape(1, _D))
    p2 = _sc_agg(h, src, dst)
    pred = _tc_post(p2, cpart, hr2, batchp.reshape(_NB, 1, _BR),
                    W2l, Wc1, bc1.reshape(1, _G), Wc2, bc2.reshape(1, _G),
                    Wc3, bc3.reshape(1, 1))
    return pred


# speedup vs baseline: 11.0488x; 1.0022x over previous
import functools

import jax
import jax.numpy as jnp
from jax import lax
from jax.experimental import pallas as pl
from jax.experimental.pallas import tpu as pltpu
from jax.experimental.pallas import tpu_sc as plsc

_N = 10000
_E = 320000
_D = 128
_G = 64

_NP = 10240
_NB = 10
_BR = _NP // _NB
_NC = 2
_NS = 16
_NW = _NC * _NS
_EPW = _E // _NW
_K = 80
_NCHUNK = _EPW // _K
_RB = _NP // _NS
_ZR = 64
_NZ = _RB // _ZR

_HI = jax.lax.Precision.HIGHEST


def _tc_pre_body(x_ref, w_ref, b_ref, o_ref):
    o_ref[...] = jnp.dot(x_ref[...], w_ref[...],
                         preferred_element_type=jnp.float32) + b_ref[...]


def _tc_lin(x, w, b):
    return pl.pallas_call(
        _tc_pre_body,
        grid=(_NB,),
        in_specs=[
            pl.BlockSpec((_BR, _D), lambda i: (i, 0)),
            pl.BlockSpec((_D, _D), lambda i: (0, 0)),
            pl.BlockSpec((1, _D), lambda i: (0, 0)),
        ],
        out_specs=pl.BlockSpec((_BR, _D), lambda i: (i, 0)),
        out_shape=jax.ShapeDtypeStruct((_NP, _D), jnp.float32),
    )(x, w, b)


def _tc_mid_body(p_ref, c_ref, xr_ref, g_ref, bt_ref, w1l_ref, h_ref):
    cnt = jnp.sum(c_ref[0] + c_ref[1], axis=1, keepdims=True) * (1.0 / _D)
    mean1 = (p_ref[0] + p_ref[1]) / jnp.maximum(cnt, 1.0)
    h = jnp.maximum(jnp.dot(mean1, w1l_ref[...],
                            preferred_element_type=jnp.float32)
                    + xr_ref[...], 0.0)
    mu = jnp.mean(h, axis=1, keepdims=True)
    d = h - mu
    var = jnp.mean(d * d, axis=1, keepdims=True)
    h_ref[...] = d / jnp.sqrt(var + 1e-5) * g_ref[...] + bt_ref[...]


def _tc_mid(p1, c, xr1, gamma, beta, w1l):
    return pl.pallas_call(
        _tc_mid_body,
        grid=(_NB,),
        in_specs=[
            pl.BlockSpec((_NC, _BR, _D), lambda i: (0, i, 0)),
            pl.BlockSpec((_NC, _BR, _D), lambda i: (0, i, 0)),
            pl.BlockSpec((_BR, _D), lambda i: (i, 0)),
            pl.BlockSpec((1, _D), lambda i: (0, 0)),
            pl.BlockSpec((1, _D), lambda i: (0, 0)),
            pl.BlockSpec((_D, _D), lambda i: (0, 0)),
        ],
        out_specs=pl.BlockSpec((_BR, _D), lambda i: (i, 0)),
        out_shape=jax.ShapeDtypeStruct((_NP, _D), jnp.float32),
    )(p1, c, xr1, gamma, beta, w1l)


def _tc_post_body(p_ref, c_ref, hr_ref, b_ref, w2l_ref, wc1_ref, bc1_ref,
                  wc2_ref, bc2_ref, wc3_ref, bc3_ref, out_ref, acc_ref):
    i = pl.program_id(0)

    @pl.when(i == 0)
    def _():
        acc_ref[...] = jnp.zeros_like(acc_ref)

    cnt = jnp.sum(c_ref[0] + c_ref[1], axis=1, keepdims=True) * (1.0 / _D)
    mean2 = (p_ref[0] + p_ref[1]) / jnp.maximum(cnt, 1.0)
    h2 = jnp.maximum(jnp.dot(mean2, w2l_ref[...],
                             preferred_element_type=jnp.float32)
                     + hr_ref[...], 0.0)
    seg = b_ref[0]
    onehot_t = (lax.broadcasted_iota(jnp.int32, (_G, _BR), 0) == seg
                ).astype(jnp.float32)
    acc_ref[...] += jnp.dot(onehot_t, h2, precision=_HI,
                            preferred_element_type=jnp.float32)

    @pl.when(i == _NB - 1)
    def _():
        pooled = acc_ref[...]
        z = jnp.maximum(jnp.dot(pooled, wc1_ref[...],
                                preferred_element_type=jnp.float32)
                        + bc1_ref[...], 0.0)
        z = jnp.maximum(jnp.dot(z, wc2_ref[...],
                                preferred_element_type=jnp.float32)
                        + bc2_ref[...], 0.0)
        out_ref[...] = jnp.dot(z, wc3_ref[...],
                               preferred_element_type=jnp.float32) + bc3_ref[...]


def _tc_post(p2, c, hr2, batch3, w2l, wc1, bc1, wc2, bc2, wc3, bc3):
    return pl.pallas_call(
        _tc_post_body,
        grid=(_NB,),
        in_specs=[
            pl.BlockSpec((_NC, _BR, _D), lambda i: (0, i, 0)),
            pl.BlockSpec((_NC, _BR, _D), lambda i: (0, i, 0)),
            pl.BlockSpec((_BR, _D), lambda i: (i, 0)),
            pl.BlockSpec((1, 1, _BR), lambda i: (i, 0, 0)),
            pl.BlockSpec((_D, _D), lambda i: (0, 0)),
            pl.BlockSpec((_D, _G), lambda i: (0, 0)),
            pl.BlockSpec((1, _G), lambda i: (0, 0)),
            pl.BlockSpec((_G, _G), lambda i: (0, 0)),
            pl.BlockSpec((1, _G), lambda i: (0, 0)),
            pl.BlockSpec((_G, 1), lambda i: (0, 0)),
            pl.BlockSpec((1, 1), lambda i: (0, 0)),
        ],
        out_specs=pl.BlockSpec((_G, 1), lambda i: (0, 0)),
        out_shape=jax.ShapeDtypeStruct((_G, 1), jnp.float32),
        scratch_shapes=[pltpu.VMEM((_G, _D), jnp.float32)],
    )(p2, c, hr2, batch3, w2l, wc1, bc1, wc2, bc2, wc3, bc3)


def _sc_cnt(dst):
    mesh = plsc.VectorSubcoreMesh(core_axis_name="c", subcore_axis_name="s")

    @functools.partial(
        pl.kernel,
        out_type=jax.ShapeDtypeStruct((_NC, _NP, _D), jnp.float32),
        mesh=mesh,
        scratch_types=[
            pltpu.VMEM_SHARED((_NP, _D), jnp.float32),
            pltpu.VMEM((4, _K), jnp.int32),
            pltpu.VMEM((_K, _D), jnp.float32),
            pltpu.VMEM((_ZR, _D), jnp.float32),
            pltpu.SemaphoreType.DMA,
            pltpu.SemaphoreType.DMA,
            pltpu.SemaphoreType.DMA,
            pltpu.SemaphoreType.DMA,
            pltpu.SemaphoreType.DMA,
            pltpu.SemaphoreType.DMA,
        ],
    )
    def k(dst_hbm, c_hbm, acc_sh, dstb, ones_v, zrows_v,
          isem0, isem1, isem2, isem3, ssem0, ssem1):
        c = lax.axis_index("c")
        s = lax.axis_index("s")
        wid = c * _NS + s
        zero16 = jnp.zeros((16,), jnp.float32)
        one16 = jnp.ones((16,), jnp.float32)
        isems = (isem0, isem1, isem2, isem3)
        ssems = (ssem0, ssem1)
        ebase = wid * _EPW

        def issue_idx(chunk_i, sl):
            off = ebase + chunk_i * _K
            pltpu.async_copy(dst_hbm.at[pl.ds(off, _K)], dstb.at[sl], isems[sl])

        def wait_idx(sl):
            pltpu.make_async_copy(dst_hbm.at[pl.ds(0, _K)], dstb.at[sl],
                                  isems[sl]).wait()

        def issue_scatter(sl, sm):
            pltpu.async_copy(ones_v, acc_sh.at[dstb.at[sl]], ssems[sm],
                             add=True)

        def wait_scatter(sl, sm):
            pltpu.make_async_copy(ones_v, acc_sh.at[dstb.at[sl]],
                                  ssems[sm]).wait()

        issue_idx(0, 0)
        issue_idx(1, 1)

        def fill(i, carry):
            for j in range(_D // 16):
                zrows_v[i, pl.ds(j * 16, 16)] = zero16
                ones_v[i, pl.ds(j * 16, 16)] = one16
            return carry

        lax.fori_loop(0, _ZR, fill, 0)

        def fill_ones(i, carry):
            for j in range(_D // 16):
                ones_v[i, pl.ds(j * 16, 16)] = one16
            return carry

        lax.fori_loop(_ZR, _K, fill_ones, 0)

        base_r = s * _RB

        def zero_spmem(r, carry):
            pltpu.sync_copy(zrows_v, acc_sh.at[pl.ds(base_r + r * _ZR, _ZR)])
            return carry

        lax.fori_loop(0, _NZ, zero_spmem, 0)
        plsc.subcore_barrier()

        issue_idx(2, 2)
        wait_idx(0)
        issue_scatter(0, 0)
        issue_idx(3, 3)
        wait_idx(1)
        issue_scatter(1, 1)

        def step(i, sl, sm):
            sl2 = (sl + 2) % 4
            wait_scatter(sl2, sm)
            issue_idx(jnp.minimum(i + 2, _NCHUNK - 1), sl2)
            wait_idx(sl)
            issue_scatter(sl, sm)

        def outer(j, carry):
            i = 4 * j + 2
            step(i, 2, 0)
            step(i + 1, 3, 1)
            step(i + 2, 0, 0)
            step(i + 3, 1, 1)
            return carry

        lax.fori_loop(0, 30, outer, 0)

        wait_scatter(0, 0)
        issue_idx(_NCHUNK - 1, 0)
        wait_idx(2)
        issue_scatter(2, 0)
        wait_scatter(1, 1)
        wait_idx(3)
        issue_scatter(3, 1)
        wait_scatter(2, 0)
        wait_idx(0)
        issue_scatter(0, 0)

        wait_scatter(3, 1)
        wait_scatter(0, 0)
        plsc.subcore_barrier()

        pltpu.sync_copy(acc_sh.at[pl.ds(base_r, _RB)],
                        c_hbm.at[c, pl.ds(base_r, _RB)])

    return k(dst)


def _sc_agg(a, src, dst):
    mesh = plsc.VectorSubcoreMesh(core_axis_name="c", subcore_axis_name="s")

    @functools.partial(
        pl.kernel,
        out_type=jax.ShapeDtypeStruct((_NC, _NP, _D), jnp.float32),
        mesh=mesh,
        scratch_types=[
            pltpu.VMEM_SHARED((_NP, _D), jnp.float32),
            pltpu.VMEM((6, _K), jnp.int32),
            pltpu.VMEM((6, _K), jnp.int32),
            pltpu.VMEM((3, _K, _D), jnp.float32),
            pltpu.VMEM((_ZR, _D), jnp.float32),
            pltpu.SemaphoreType.DMA,
            pltpu.SemaphoreType.DMA,
            pltpu.SemaphoreType.DMA,
            pltpu.SemaphoreType.DMA,
            pltpu.SemaphoreType.DMA,
            pltpu.SemaphoreType.DMA,
            pltpu.SemaphoreType.DMA,
            pltpu.SemaphoreType.DMA,
            pltpu.SemaphoreType.DMA,
            pltpu.SemaphoreType.DMA,
            pltpu.SemaphoreType.DMA,
            pltpu.SemaphoreType.DMA,
        ],
    )
    def k(a_hbm, src_hbm, dst_hbm, p_hbm, acc_sh, srcb, dstb, rowsb, zrows_v,
          isem0, isem1, isem2, isem3, isem4, isem5,
          gsem0, gsem1, gsem2, ssem0, ssem1, ssem2):
        c = lax.axis_index("c")
        s = lax.axis_index("s")
        wid = c * _NS + s
        zero16 = jnp.zeros((16,), jnp.float32)
        isems = (isem0, isem1, isem2, isem3, isem4, isem5)
        gsems = (gsem0, gsem1, gsem2)
        ssems = (ssem0, ssem1, ssem2)
        ebase = wid * _EPW

        def issue_idx(chunk_i, sl):
            off = ebase + chunk_i * _K
            pltpu.async_copy(src_hbm.at[pl.ds(off, _K)], srcb.at[sl], isems[sl])
            pltpu.async_copy(dst_hbm.at[pl.ds(off, _K)], dstb.at[sl], isems[sl])

        def wait_idx(sl):
            pltpu.make_async_copy(src_hbm.at[pl.ds(0, _K)], srcb.at[sl],
                                  isems[sl]).wait()
            pltpu.make_async_copy(dst_hbm.at[pl.ds(0, _K)], dstb.at[sl],
                                  isems[sl]).wait()

        def issue_gather(sl, rb):
            pltpu.async_copy(a_hbm.at[srcb.at[sl]], rowsb.at[rb], gsems[rb])

        def wait_gather(sl, rb):
            pltpu.make_async_copy(a_hbm.at[srcb.at[sl]], rowsb.at[rb],
                                  gsems[rb]).wait()

        def issue_scatter(sl, rb, sm):
            pltpu.async_copy(rowsb.at[rb], acc_sh.at[dstb.at[sl]],
                             ssems[sm], add=True)

        def wait_scatter(sl, rb, sm):
            pltpu.make_async_copy(rowsb.at[rb], acc_sh.at[dstb.at[sl]],
                                  ssems[sm]).wait()

        issue_idx(0, 0)
        issue_idx(1, 1)
        issue_idx(2, 2)
        issue_idx(3, 3)

        def fill(i, carry):
            for j in range(_D // 16):
                zrows_v[i, pl.ds(j * 16, 16)] = zero16
            return carry

        lax.fori_loop(0, _ZR, fill, 0)

        base_r = s * _RB

        def zero_spmem(r, carry):
            pltpu.sync_copy(zrows_v, acc_sh.at[pl.ds(base_r + r * _ZR, _ZR)])
            return carry

        lax.fori_loop(0, _NZ, zero_spmem, 0)
        plsc.subcore_barrier()

        issue_idx(4, 4)
        wait_idx(0)
        issue_gather(0, 0)
        wait_idx(1)
        issue_gather(1, 1)
        wait_gather(0, 0)
        issue_scatter(0, 0, 0)
        issue_idx(5, 5)
        wait_idx(2)
        issue_gather(2, 2)
        wait_gather(1, 1)
        issue_scatter(1, 1, 1)

        def step(i, sl, rb, sm):
            slm2 = (sl + 4) % 6
            rbm2 = (rb + 1) % 3
            sl1 = (sl + 1) % 6
            wait_scatter(slm2, rbm2, rbm2)
            issue_idx(jnp.minimum(i + 4, _NCHUNK - 1), slm2)
            wait_idx(sl1)
            issue_gather(sl1, rbm2)
            wait_gather(sl, rb)
            issue_scatter(sl, rb, sm)

        def outer(j, carry):
            i = 6 * j + 2
            step(i, 2, 2, 2)
            step(i + 1, 3, 0, 0)
            step(i + 2, 4, 1, 1)
            step(i + 3, 5, 2, 2)
            step(i + 4, 0, 0, 0)
            step(i + 5, 1, 1, 1)
            return carry

        lax.fori_loop(0, 20, outer, 0)

        wait_scatter(0, 0, 0)
        issue_idx(_NCHUNK - 1, 0)
        wait_idx(3)
        issue_gather(3, 0)
        wait_gather(2, 2)
        issue_scatter(2, 2, 2)
        wait_scatter(1, 1, 1)
        issue_idx(_NCHUNK - 1, 1)
        wait_idx(4)
        issue_gather(4, 1)
        wait_gather(3, 0)
        issue_scatter(3, 0, 0)
        wait_scatter(2, 2, 2)
        wait_gather(4, 1)
        issue_scatter(4, 1, 1)

        wait_scatter(3, 0, 0)
        wait_scatter(4, 1, 1)
        wait_idx(5)
        wait_idx(0)
        wait_idx(1)
        plsc.subcore_barrier()

        pltpu.sync_copy(acc_sh.at[pl.ds(base_r, _RB)],
                        p_hbm.at[c, pl.ds(base_r, _RB)])

    return k(a, src, dst)


def kernel(x, edge_index, batch, W1l, W1r, b1, gamma, beta, W2l, W2r, b2,
           Wc1, bc1, Wc2, bc2, Wc3, bc3):
    src = edge_index[0]
    dst = edge_index[1]
    xp = jnp.pad(x, ((0, _NP - _N), (0, 0)))
    batchp = jnp.pad(batch, (0, _NP - _N), constant_values=_G)

    xr1 = _tc_lin(xp, W1r, b1.reshape(1, _D))
    cpart = _sc_cnt(dst)
    p1 = _sc_agg(xp, src, dst)
    h = _tc_mid(p1, cpart, xr1, gamma.reshape(1, _D), beta.reshape(1, _D),
                W1l)
    hr2 = _tc_lin(h, W2r, b2.reshape(1, _D))
    p2 = _sc_agg(h, src, dst)
    pred = _tc_post(p2, cpart, hr2, batchp.reshape(_NB, 1, _BR),
                    W2l, Wc1, bc1.reshape(1, _G), Wc2, bc2.reshape(1, _G),
                    Wc3, bc3.reshape(1, 1))
    return pred
